# Initial kernel scaffold; baseline (speedup 1.0000x reference)
import functools

import jax
import jax.numpy as jnp
from jax import lax
from jax.experimental import pallas as pl
from jax.experimental.pallas import tpu as pltpu
from jax.experimental.pallas import tpu_sc as plsc

N = 10000
E = 320000
G = 16
D_NODE = 128
D_EDGE = 16
D_U = 16
MSG = 32

NC = 2
NS = 16
NW = NC * NS
ROWS_PER_W = 80
IDX_ROWS = NW * ROWS_PER_W
E_PAD = IDX_ROWS * 128
CH = 8
N_CHUNKS = ROWS_PER_W // CH
N_ACC = N + 16
ACC_W = 48
ROWS_PER_TILE = N_ACC // NS
NBLK = 1000
EBLK = 2048

_f32 = jnp.float32


def _tc1_body(xb, bb, u, We1, be1, Wn1a, bn1a, trow, tcol):
    xv = xb[...]
    b = bb[0, 0, :]
    oh = (b[:, None] == lax.broadcasted_iota(jnp.int32, (NBLK, G), 1)).astype(_f32)
    w = We1[...]
    ub1 = jnp.dot(u[...], w[2 * D_NODE + D_EDGE:, :], preferred_element_type=_f32)
    rowp = (jnp.dot(xv, w[:D_NODE, :], preferred_element_type=_f32)
            + jnp.dot(oh, ub1, preferred_element_type=_f32) + be1[0])
    xm = jnp.dot(xv, Wn1a[:D_NODE, :], preferred_element_type=_f32) + bn1a[0]
    trow[:, 0:MSG] = rowp
    trow[:, MSG:2 * MSG] = xm
    tcol[...] = jnp.dot(xv, w[D_NODE:2 * D_NODE, :], preferred_element_type=_f32)


def _tc1(x, batch3, u, We1, be1, Wn1a, bn1a):
    grid = N // NBLK
    return pl.pallas_call(
        _tc1_body,
        grid=(grid,),
        in_specs=[
            pl.BlockSpec((NBLK, D_NODE), lambda i: (i, 0)),
            pl.BlockSpec((1, 1, NBLK), lambda i: (i, 0, 0)),
            pl.BlockSpec(u.shape, lambda i: (0, 0)),
            pl.BlockSpec(We1.shape, lambda i: (0, 0)),
            pl.BlockSpec(be1.shape, lambda i: (0, 0)),
            pl.BlockSpec(Wn1a.shape, lambda i: (0, 0)),
            pl.BlockSpec(bn1a.shape, lambda i: (0, 0)),
        ],
        out_specs=[
            pl.BlockSpec((NBLK, 2 * MSG), lambda i: (i, 0)),
            pl.BlockSpec((NBLK, MSG), lambda i: (i, 0)),
        ],
        out_shape=[
            jax.ShapeDtypeStruct((N, 2 * MSG), _f32),
            jax.ShapeDtypeStruct((N, MSG), _f32),
        ],
    )(x, batch3, u, We1, be1, Wn1a, bn1a)


def _sc_gather_body(trow, tcol, ridx, cidx, grow, gcol,
                    idxr, idxc, bufr, bufc, semi, semg, semw):
    c = lax.axis_index("c")
    s = lax.axis_index("s")
    wid = s * NC + c
    rbase = wid * ROWS_PER_W

    def chunk(ci, carry):
        r0 = rbase + ci * CH
        e0 = r0 * 128
        cp1 = pltpu.async_copy(ridx.at[pl.ds(r0, CH)], idxr, semi)
        cp2 = pltpu.async_copy(cidx.at[pl.ds(r0, CH)], idxc, semi)
        cp1.wait()
        cp2.wait()
        cps = []
        for j in range(CH):
            cps.append(pltpu.async_copy(
                trow.at[idxr.at[j]], bufr.at[pl.ds(j * 128, 128)], semg))
            cps.append(pltpu.async_copy(
                tcol.at[idxc.at[j]], bufc.at[pl.ds(j * 128, 128)], semg))
        for cp in cps:
            cp.wait()
        w1 = pltpu.async_copy(bufr, grow.at[pl.ds(e0, CH * 128)], semw)
        w2 = pltpu.async_copy(bufc, gcol.at[pl.ds(e0, CH * 128)], semw)
        w1.wait()
        w2.wait()
        return carry

    lax.fori_loop(0, N_CHUNKS, chunk, 0)


def _sc_gather(trow, tcol, ridx, cidx):
    mesh = plsc.VectorSubcoreMesh(core_axis_name="c", subcore_axis_name="s",
                                  num_cores=NC, num_subcores=NS)
    fn = pl.kernel(
        _sc_gather_body,
        out_type=[
            jax.ShapeDtypeStruct((E_PAD, 2 * MSG), _f32),
            jax.ShapeDtypeStruct((E_PAD, MSG), _f32),
        ],
        mesh=mesh,
        scratch_types=[
            pltpu.VMEM((CH, 128), jnp.int32),
            pltpu.VMEM((CH, 128), jnp.int32),
            pltpu.VMEM((CH * 128, 2 * MSG), _f32),
            pltpu.VMEM((CH * 128, MSG), _f32),
            pltpu.SemaphoreType.DMA,
            pltpu.SemaphoreType.DMA,
            pltpu.SemaphoreType.DMA,
        ],
    )
    return fn(trow, tcol, ridx, cidx)


def _tc2_body(growb, gcolb, eab, We1, We2, be2, Wn1a, Wn1b, bn1b, en_out, m_out):
    gr = growb[...]
    eap = jnp.dot(eab[...], We1[2 * D_NODE:2 * D_NODE + D_EDGE, :],
                  preferred_element_type=_f32)
    h1 = jnp.maximum(gr[:, 0:MSG] + gcolb[...] + eap, 0.0)
    en = jnp.dot(h1, We2[...], preferred_element_type=_f32) + be2[0]
    en_out[...] = en
    mh = jnp.maximum(
        gr[:, MSG:2 * MSG]
        + jnp.dot(en, Wn1a[D_NODE:D_NODE + D_EDGE, :], preferred_element_type=_f32),
        0.0)
    m_out[...] = jnp.dot(mh, Wn1b[...], preferred_element_type=_f32) + bn1b[0]


def _tc2(grow, gcol, ea_pad, We1, We2, be2, Wn1a, Wn1b, bn1b):
    grid = E_PAD // EBLK
    return pl.pallas_call(
        _tc2_body,
        grid=(grid,),
        in_specs=[
            pl.BlockSpec((EBLK, 2 * MSG), lambda i: (i, 0)),
            pl.BlockSpec((EBLK, MSG), lambda i: (i, 0)),
            pl.BlockSpec((EBLK, D_EDGE), lambda i: (i, 0)),
            pl.BlockSpec(We1.shape, lambda i: (0, 0)),
            pl.BlockSpec(We2.shape, lambda i: (0, 0)),
            pl.BlockSpec(be2.shape, lambda i: (0, 0)),
            pl.BlockSpec(Wn1a.shape, lambda i: (0, 0)),
            pl.BlockSpec(Wn1b.shape, lambda i: (0, 0)),
            pl.BlockSpec(bn1b.shape, lambda i: (0, 0)),
        ],
        out_specs=[
            pl.BlockSpec((EBLK, D_EDGE), lambda i: (i, 0)),
            pl.BlockSpec((EBLK, MSG), lambda i: (i, 0)),
        ],
        out_shape=[
            jax.ShapeDtypeStruct((E_PAD, D_EDGE), _f32),
            jax.ShapeDtypeStruct((E_PAD, MSG), _f32),
        ],
    )(grow, gcol, ea_pad, We1, We2, be2, Wn1a, Wn1b, bn1b)


def _sc_scatter_body(m_hbm, cidx, out_hbm, acc, mbuf, idx8, zbuf,
                     semz, semi, semm, sems, semo):
    c = lax.axis_index("c")
    s = lax.axis_index("s")
    wid = s * NC + c

    def zfill(i, carry):
        zbuf[i, pl.ds(0, 16)] = jnp.zeros((16,), _f32)
        zbuf[i, pl.ds(16, 16)] = jnp.zeros((16,), _f32)
        zbuf[i, pl.ds(32, 16)] = jnp.zeros((16,), _f32)
        return carry

    lax.fori_loop(0, ROWS_PER_TILE // 2, zfill, 0)
    row0 = s * ROWS_PER_TILE
    pltpu.async_copy(zbuf, acc.at[pl.ds(row0, ROWS_PER_TILE // 2)], semz).wait()
    pltpu.async_copy(
        zbuf, acc.at[pl.ds(row0 + ROWS_PER_TILE // 2, ROWS_PER_TILE // 2)],
        semz).wait()

    def ofill(i, carry):
        mbuf[i, pl.ds(MSG, 16)] = jnp.full((16,), 1.0, _f32)
        return carry

    lax.fori_loop(0, CH * 128, ofill, 0)
    plsc.subcore_barrier()

    def chunk(ci, carry):
        r0 = wid * ROWS_PER_W + ci * CH
        e0 = r0 * 128
        cp1 = pltpu.async_copy(cidx.at[pl.ds(r0, CH)], idx8, semi)
        cp2 = pltpu.async_copy(m_hbm.at[pl.ds(e0, CH * 128)],
                               mbuf.at[:, pl.ds(0, MSG)], semm)
        cp1.wait()
        cp2.wait()
        cps = []
        for j in range(CH):
            cps.append(pltpu.async_copy(
                mbuf.at[pl.ds(j * 128, 128)], acc.at[idx8.at[j]], sems,
                add=True))
        for cp in cps:
            cp.wait()
        return carry

    lax.fori_loop(0, N_CHUNKS, chunk, 0)
    plsc.subcore_barrier()
    pltpu.async_copy(acc.at[pl.ds(row0, ROWS_PER_TILE)],
                     out_hbm.at[c, pl.ds(row0, ROWS_PER_TILE)], semo).wait()


def _sc_scatter(m_pad, cidx_s):
    mesh = plsc.VectorSubcoreMesh(core_axis_name="c", subcore_axis_name="s",
                                  num_cores=NC, num_subcores=NS)
    fn = pl.kernel(
        _sc_scatter_body,
        out_type=jax.ShapeDtypeStruct((NC, N_ACC, ACC_W), _f32),
        mesh=mesh,
        scratch_types=[
            pltpu.VMEM_SHARED((N_ACC, ACC_W), _f32),
            pltpu.VMEM((CH * 128, ACC_W), _f32),
            pltpu.VMEM((CH, 128), jnp.int32),
            pltpu.VMEM((ROWS_PER_TILE // 2, ACC_W), _f32),
            pltpu.SemaphoreType.DMA,
            pltpu.SemaphoreType.DMA,
            pltpu.SemaphoreType.DMA,
            pltpu.SemaphoreType.DMA,
            pltpu.SemaphoreType.DMA,
        ],
    )
    return fn(m_pad, cidx_s)


def _tc3_body(xb, pb, bb, u, Wn2a, bn2a, Wn2b, bn2b, Wg1, bg1, Wg2, bg2,
              xn_out, un_out, xsum, nct):
    i = pl.program_id(0)
    p = pb[...]
    ssum = p[0] + p[1]
    agg = ssum[:, 0:MSG] / jnp.maximum(ssum[:, MSG:MSG + 1], 1.0)
    b = bb[0, 0, :]
    oh = (b[:, None] == lax.broadcasted_iota(jnp.int32, (NBLK, G), 1)).astype(_f32)
    wa = Wn2a[...]
    ub2 = jnp.dot(u[...], wa[D_NODE + MSG:, :], preferred_element_type=_f32)
    h = jnp.maximum(
        jnp.dot(xb[...], wa[:D_NODE, :], preferred_element_type=_f32)
        + jnp.dot(agg, wa[D_NODE:D_NODE + MSG, :], preferred_element_type=_f32)
        + jnp.dot(oh, ub2, preferred_element_type=_f32) + bn2a[0],
        0.0)
    xn = jnp.dot(h, Wn2b[...], preferred_element_type=_f32) + bn2b[0]
    xn_out[...] = xn

    @pl.when(i == 0)
    def _init():
        xsum[...] = jnp.zeros((G, D_NODE), _f32)
        nct[...] = jnp.zeros((G, D_NODE), _f32)

    dn = (((0,), (0,)), ((), ()))
    xsum[...] += lax.dot_general(oh, xn, dn, preferred_element_type=_f32)
    nct[...] += lax.dot_general(oh, jnp.ones((NBLK, D_NODE), _f32), dn,
                                preferred_element_type=_f32)

    @pl.when(i == (N // NBLK) - 1)
    def _fin():
        xmean = xsum[...] / jnp.maximum(nct[...], 1.0)
        uu = u[...]
        gh = jnp.maximum(
            jnp.dot(uu, Wg1[:D_U, :], preferred_element_type=_f32)
            + jnp.dot(xmean, Wg1[D_U:, :], preferred_element_type=_f32)
            + bg1[0],
            0.0)
        un_out[...] = jnp.dot(gh, Wg2[...], preferred_element_type=_f32) + bg2[0]


def _tc3(x, parts, batch3, u, Wn2a, bn2a, Wn2b, bn2b, Wg1, bg1, Wg2, bg2):
    grid = N // NBLK
    return pl.pallas_call(
        _tc3_body,
        grid=(grid,),
        in_specs=[
            pl.BlockSpec((NBLK, D_NODE), lambda i: (i, 0)),
            pl.BlockSpec((NC, NBLK, ACC_W), lambda i: (0, i, 0)),
            pl.BlockSpec((1, 1, NBLK), lambda i: (i, 0, 0)),
            pl.BlockSpec(u.shape, lambda i: (0, 0)),
            pl.BlockSpec(Wn2a.shape, lambda i: (0, 0)),
            pl.BlockSpec(bn2a.shape, lambda i: (0, 0)),
            pl.BlockSpec(Wn2b.shape, lambda i: (0, 0)),
            pl.BlockSpec(bn2b.shape, lambda i: (0, 0)),
            pl.BlockSpec(Wg1.shape, lambda i: (0, 0)),
            pl.BlockSpec(bg1.shape, lambda i: (0, 0)),
            pl.BlockSpec(Wg2.shape, lambda i: (0, 0)),
            pl.BlockSpec(bg2.shape, lambda i: (0, 0)),
        ],
        out_specs=[
            pl.BlockSpec((NBLK, D_NODE), lambda i: (i, 0)),
            pl.BlockSpec((G, D_U), lambda i: (0, 0)),
        ],
        out_shape=[
            jax.ShapeDtypeStruct((N, D_NODE), _f32),
            jax.ShapeDtypeStruct((G, D_U), _f32),
        ],
        scratch_shapes=[
            pltpu.VMEM((G, D_NODE), _f32),
            pltpu.VMEM((G, D_NODE), _f32),
        ],
    )(x, parts, batch3, u, Wn2a, bn2a, Wn2b, bn2b, Wg1, bg1, Wg2, bg2)


def kernel(x, edge_index, edge_attr, u, batch,
           We1, be1, We2, be2,
           Wn1a, bn1a, Wn1b, bn1b,
           Wn2a, bn2a, Wn2b, bn2b,
           Wg1, bg1, Wg2, bg2):
    row = edge_index[0]
    col = edge_index[1]
    pad = E_PAD - E
    ridx = jnp.pad(row, (0, pad)).reshape(IDX_ROWS, 128)
    cidx_g = jnp.pad(col, (0, pad)).reshape(IDX_ROWS, 128)
    cidx_s = jnp.pad(col, (0, pad), constant_values=N).reshape(IDX_ROWS, 128)
    ea_pad = jnp.pad(edge_attr, ((0, pad), (0, 0)))
    batch3 = batch.reshape(N // NBLK, 1, NBLK)
    be1_ = be1.reshape(1, -1)
    be2_ = be2.reshape(1, -1)
    bn1a_ = bn1a.reshape(1, -1)
    bn1b_ = bn1b.reshape(1, -1)
    bn2a_ = bn2a.reshape(1, -1)
    bn2b_ = bn2b.reshape(1, -1)
    bg1_ = bg1.reshape(1, -1)
    bg2_ = bg2.reshape(1, -1)

    trow, tcol = _tc1(x, batch3, u, We1, be1_, Wn1a, bn1a_)
    grow, gcol = _sc_gather(trow, tcol, ridx, cidx_g)
    en_pad, m_pad = _tc2(grow, gcol, ea_pad, We1, We2, be2_, Wn1a, Wn1b, bn1b_)
    parts = _sc_scatter(m_pad, cidx_s)
    x_new, u_new = _tc3(x, parts, batch3, u,
                        Wn2a, bn2a_, Wn2b, bn2b_, Wg1, bg1_, Wg2, bg2_)
    edge_new = en_pad[:E]
    return (x_new, edge_new, u_new)


# speedup vs baseline: 4.4139x; 4.4139x over previous
import functools

import jax
import jax.numpy as jnp
from jax import lax
from jax.experimental import pallas as pl
from jax.experimental.pallas import tpu as pltpu
from jax.experimental.pallas import tpu_sc as plsc

N = 10000
E = 320000
G = 16
D_NODE = 128
D_EDGE = 16
D_U = 16
MSG = 32

NC = 2
NS = 16
NW = NC * NS
ROWS_PER_W = 80
IDX_ROWS = NW * ROWS_PER_W
E_PAD = IDX_ROWS * 128
CH = 8
N_CHUNKS = ROWS_PER_W // CH
N_ACC = N + 16
ACC_W = 48
ROWS_PER_TILE = N_ACC // NS
NBLK = 1000
EBLK = 2048

_f32 = jnp.float32


def _tc1_body(xb, bb, u, We1, be1, Wn1a, bn1a, trow, tcol):
    xv = xb[...]
    b = bb[0, 0, :]
    oh = (b[:, None] == lax.broadcasted_iota(jnp.int32, (NBLK, G), 1)).astype(_f32)
    w = We1[...]
    ub1 = jnp.dot(u[...], w[2 * D_NODE + D_EDGE:, :], preferred_element_type=_f32)
    rowp = (jnp.dot(xv, w[:D_NODE, :], preferred_element_type=_f32)
            + jnp.dot(oh, ub1, preferred_element_type=_f32) + be1[0])
    xm = jnp.dot(xv, Wn1a[:D_NODE, :], preferred_element_type=_f32) + bn1a[0]
    trow[:, 0:MSG] = rowp
    trow[:, MSG:2 * MSG] = xm
    tcol[...] = jnp.dot(xv, w[D_NODE:2 * D_NODE, :], preferred_element_type=_f32)


def _tc1(x, batch3, u, We1, be1, Wn1a, bn1a):
    grid = N // NBLK
    return pl.pallas_call(
        _tc1_body,
        grid=(grid,),
        in_specs=[
            pl.BlockSpec((NBLK, D_NODE), lambda i: (i, 0)),
            pl.BlockSpec((1, 1, NBLK), lambda i: (i, 0, 0)),
            pl.BlockSpec(u.shape, lambda i: (0, 0)),
            pl.BlockSpec(We1.shape, lambda i: (0, 0)),
            pl.BlockSpec(be1.shape, lambda i: (0, 0)),
            pl.BlockSpec(Wn1a.shape, lambda i: (0, 0)),
            pl.BlockSpec(bn1a.shape, lambda i: (0, 0)),
        ],
        out_specs=[
            pl.BlockSpec((NBLK, 2 * MSG), lambda i: (i, 0)),
            pl.BlockSpec((NBLK, MSG), lambda i: (i, 0)),
        ],
        out_shape=[
            jax.ShapeDtypeStruct((N, 2 * MSG), _f32),
            jax.ShapeDtypeStruct((N, MSG), _f32),
        ],
    )(x, batch3, u, We1, be1, Wn1a, bn1a)


def _sc_gather_body(trow, tcol, ridx, cidx, grow, gcol,
                    idxr, idxc, bufr, bufc, semi, semg, semw):
    c = lax.axis_index("c")
    s = lax.axis_index("s")
    wid = s * NC + c
    rbase = wid * ROWS_PER_W

    def chunk(ci, carry):
        r0 = rbase + ci * CH
        e0 = r0 * 128
        cp1 = pltpu.async_copy(ridx.at[pl.ds(r0, CH)], idxr, semi)
        cp2 = pltpu.async_copy(cidx.at[pl.ds(r0, CH)], idxc, semi)
        cp1.wait()
        cp2.wait()
        cps = []
        for j in range(CH):
            cps.append(pltpu.async_copy(
                trow.at[idxr.at[j]], bufr.at[pl.ds(j * 128, 128)], semg))
            cps.append(pltpu.async_copy(
                tcol.at[idxc.at[j]], bufc.at[pl.ds(j * 128, 128)], semg))
        for cp in cps:
            cp.wait()
        w1 = pltpu.async_copy(bufr, grow.at[pl.ds(e0, CH * 128)], semw)
        w2 = pltpu.async_copy(bufc, gcol.at[pl.ds(e0, CH * 128)], semw)
        w1.wait()
        w2.wait()
        return carry

    lax.fori_loop(0, N_CHUNKS, chunk, 0)


def _sc_gather(trow, tcol, ridx, cidx):
    mesh = plsc.VectorSubcoreMesh(core_axis_name="c", subcore_axis_name="s",
                                  num_cores=NC, num_subcores=NS)
    fn = pl.kernel(
        _sc_gather_body,
        out_type=[
            jax.ShapeDtypeStruct((E_PAD, 2 * MSG), _f32),
            jax.ShapeDtypeStruct((E_PAD, MSG), _f32),
        ],
        mesh=mesh,
        compiler_params=pltpu.CompilerParams(use_tc_tiling_on_sc=False),
        scratch_types=[
            pltpu.VMEM((CH, 128), jnp.int32),
            pltpu.VMEM((CH, 128), jnp.int32),
            pltpu.VMEM((CH * 128, 2 * MSG), _f32),
            pltpu.VMEM((CH * 128, MSG), _f32),
            pltpu.SemaphoreType.DMA,
            pltpu.SemaphoreType.DMA,
            pltpu.SemaphoreType.DMA,
        ],
    )
    return fn(trow, tcol, ridx, cidx)


def _tc2_body(growb, gcolb, eab, We1, We2, be2, Wn1a, Wn1b, bn1b, en_out, m_out):
    gr = growb[...]
    eap = jnp.dot(eab[...], We1[2 * D_NODE:2 * D_NODE + D_EDGE, :],
                  preferred_element_type=_f32)
    h1 = jnp.maximum(gr[:, 0:MSG] + gcolb[...] + eap, 0.0)
    en = jnp.dot(h1, We2[...], preferred_element_type=_f32) + be2[0]
    en_out[...] = en
    mh = jnp.maximum(
        gr[:, MSG:2 * MSG]
        + jnp.dot(en, Wn1a[D_NODE:D_NODE + D_EDGE, :], preferred_element_type=_f32),
        0.0)
    m_out[...] = jnp.dot(mh, Wn1b[...], preferred_element_type=_f32) + bn1b[0]


def _tc2(grow, gcol, ea_pad, We1, We2, be2, Wn1a, Wn1b, bn1b):
    grid = E_PAD // EBLK
    return pl.pallas_call(
        _tc2_body,
        grid=(grid,),
        in_specs=[
            pl.BlockSpec((EBLK, 2 * MSG), lambda i: (i, 0)),
            pl.BlockSpec((EBLK, MSG), lambda i: (i, 0)),
            pl.BlockSpec((EBLK, D_EDGE), lambda i: (i, 0)),
            pl.BlockSpec(We1.shape, lambda i: (0, 0)),
            pl.BlockSpec(We2.shape, lambda i: (0, 0)),
            pl.BlockSpec(be2.shape, lambda i: (0, 0)),
            pl.BlockSpec(Wn1a.shape, lambda i: (0, 0)),
            pl.BlockSpec(Wn1b.shape, lambda i: (0, 0)),
            pl.BlockSpec(bn1b.shape, lambda i: (0, 0)),
        ],
        out_specs=[
            pl.BlockSpec((EBLK, D_EDGE), lambda i: (i, 0)),
            pl.BlockSpec((EBLK, MSG), lambda i: (i, 0)),
        ],
        out_shape=[
            jax.ShapeDtypeStruct((E_PAD, D_EDGE), _f32),
            jax.ShapeDtypeStruct((E_PAD, MSG), _f32),
        ],
    )(grow, gcol, ea_pad, We1, We2, be2, Wn1a, Wn1b, bn1b)


def _sc_scatter_body(m_hbm, cidx, out_hbm, acc, mbuf, idx8, zbuf,
                     semz, semi, semm, sems, semo):
    c = lax.axis_index("c")
    s = lax.axis_index("s")
    wid = s * NC + c

    def zfill(i, carry):
        zbuf[i, pl.ds(0, 16)] = jnp.zeros((16,), _f32)
        zbuf[i, pl.ds(16, 16)] = jnp.zeros((16,), _f32)
        zbuf[i, pl.ds(32, 16)] = jnp.zeros((16,), _f32)
        return carry

    lax.fori_loop(0, ROWS_PER_TILE // 2, zfill, 0)
    row0 = s * ROWS_PER_TILE
    pltpu.async_copy(zbuf, acc.at[pl.ds(row0, ROWS_PER_TILE // 2)], semz).wait()
    pltpu.async_copy(
        zbuf, acc.at[pl.ds(row0 + ROWS_PER_TILE // 2, ROWS_PER_TILE // 2)],
        semz).wait()

    def ofill(i, carry):
        mbuf[i, pl.ds(MSG, 16)] = jnp.full((16,), 1.0, _f32)
        return carry

    lax.fori_loop(0, CH * 128, ofill, 0)
    plsc.subcore_barrier()

    def chunk(ci, carry):
        r0 = wid * ROWS_PER_W + ci * CH
        e0 = r0 * 128
        cp1 = pltpu.async_copy(cidx.at[pl.ds(r0, CH)], idx8, semi)
        cp2 = pltpu.async_copy(m_hbm.at[pl.ds(e0, CH * 128)],
                               mbuf.at[:, pl.ds(0, MSG)], semm)
        cp1.wait()
        cp2.wait()
        cps = []
        for j in range(CH):
            cps.append(pltpu.async_copy(
                mbuf.at[pl.ds(j * 128, 128)], acc.at[idx8.at[j]], sems,
                add=True))
        for cp in cps:
            cp.wait()
        return carry

    lax.fori_loop(0, N_CHUNKS, chunk, 0)
    plsc.subcore_barrier()
    pltpu.async_copy(acc.at[pl.ds(row0, ROWS_PER_TILE)],
                     out_hbm.at[c, pl.ds(row0, ROWS_PER_TILE)], semo).wait()


def _sc_scatter(m_pad, cidx_s):
    mesh = plsc.VectorSubcoreMesh(core_axis_name="c", subcore_axis_name="s",
                                  num_cores=NC, num_subcores=NS)
    fn = pl.kernel(
        _sc_scatter_body,
        out_type=jax.ShapeDtypeStruct((NC, N_ACC, ACC_W), _f32),
        mesh=mesh,
        compiler_params=pltpu.CompilerParams(use_tc_tiling_on_sc=False),
        scratch_types=[
            pltpu.VMEM_SHARED((N_ACC, ACC_W), _f32),
            pltpu.VMEM((CH * 128, ACC_W), _f32),
            pltpu.VMEM((CH, 128), jnp.int32),
            pltpu.VMEM((ROWS_PER_TILE // 2, ACC_W), _f32),
            pltpu.SemaphoreType.DMA,
            pltpu.SemaphoreType.DMA,
            pltpu.SemaphoreType.DMA,
            pltpu.SemaphoreType.DMA,
            pltpu.SemaphoreType.DMA,
        ],
    )
    return fn(m_pad, cidx_s)


def _tc3_body(xb, pb, bb, u, Wn2a, bn2a, Wn2b, bn2b, Wg1, bg1, Wg2, bg2,
              xn_out, un_out, xsum, nct):
    i = pl.program_id(0)
    p = pb[...]
    ssum = p[0] + p[1]
    agg = ssum[:, 0:MSG] / jnp.maximum(ssum[:, MSG:MSG + 1], 1.0)
    b = bb[0, 0, :]
    oh = (b[:, None] == lax.broadcasted_iota(jnp.int32, (NBLK, G), 1)).astype(_f32)
    wa = Wn2a[...]
    ub2 = jnp.dot(u[...], wa[D_NODE + MSG:, :], preferred_element_type=_f32)
    h = jnp.maximum(
        jnp.dot(xb[...], wa[:D_NODE, :], preferred_element_type=_f32)
        + jnp.dot(agg, wa[D_NODE:D_NODE + MSG, :], preferred_element_type=_f32)
        + jnp.dot(oh, ub2, preferred_element_type=_f32) + bn2a[0],
        0.0)
    xn = jnp.dot(h, Wn2b[...], preferred_element_type=_f32) + bn2b[0]
    xn_out[...] = xn

    @pl.when(i == 0)
    def _init():
        xsum[...] = jnp.zeros((G, D_NODE), _f32)
        nct[...] = jnp.zeros((G, D_NODE), _f32)

    dn = (((0,), (0,)), ((), ()))
    xsum[...] += lax.dot_general(oh, xn, dn, preferred_element_type=_f32)
    nct[...] += lax.dot_general(oh, jnp.ones((NBLK, D_NODE), _f32), dn,
                                preferred_element_type=_f32)

    @pl.when(i == (N // NBLK) - 1)
    def _fin():
        xmean = xsum[...] / jnp.maximum(nct[...], 1.0)
        uu = u[...]
        gh = jnp.maximum(
            jnp.dot(uu, Wg1[:D_U, :], preferred_element_type=_f32)
            + jnp.dot(xmean, Wg1[D_U:, :], preferred_element_type=_f32)
            + bg1[0],
            0.0)
        un_out[...] = jnp.dot(gh, Wg2[...], preferred_element_type=_f32) + bg2[0]


def _tc3(x, parts, batch3, u, Wn2a, bn2a, Wn2b, bn2b, Wg1, bg1, Wg2, bg2):
    grid = N // NBLK
    return pl.pallas_call(
        _tc3_body,
        grid=(grid,),
        in_specs=[
            pl.BlockSpec((NBLK, D_NODE), lambda i: (i, 0)),
            pl.BlockSpec((NC, NBLK, ACC_W), lambda i: (0, i, 0)),
            pl.BlockSpec((1, 1, NBLK), lambda i: (i, 0, 0)),
            pl.BlockSpec(u.shape, lambda i: (0, 0)),
            pl.BlockSpec(Wn2a.shape, lambda i: (0, 0)),
            pl.BlockSpec(bn2a.shape, lambda i: (0, 0)),
            pl.BlockSpec(Wn2b.shape, lambda i: (0, 0)),
            pl.BlockSpec(bn2b.shape, lambda i: (0, 0)),
            pl.BlockSpec(Wg1.shape, lambda i: (0, 0)),
            pl.BlockSpec(bg1.shape, lambda i: (0, 0)),
            pl.BlockSpec(Wg2.shape, lambda i: (0, 0)),
            pl.BlockSpec(bg2.shape, lambda i: (0, 0)),
        ],
        out_specs=[
            pl.BlockSpec((NBLK, D_NODE), lambda i: (i, 0)),
            pl.BlockSpec((G, D_U), lambda i: (0, 0)),
        ],
        out_shape=[
            jax.ShapeDtypeStruct((N, D_NODE), _f32),
            jax.ShapeDtypeStruct((G, D_U), _f32),
        ],
        scratch_shapes=[
            pltpu.VMEM((G, D_NODE), _f32),
            pltpu.VMEM((G, D_NODE), _f32),
        ],
    )(x, parts, batch3, u, Wn2a, bn2a, Wn2b, bn2b, Wg1, bg1, Wg2, bg2)


def kernel(x, edge_index, edge_attr, u, batch,
           We1, be1, We2, be2,
           Wn1a, bn1a, Wn1b, bn1b,
           Wn2a, bn2a, Wn2b, bn2b,
           Wg1, bg1, Wg2, bg2):
    row = edge_index[0]
    col = edge_index[1]
    pad = E_PAD - E
    ridx = jnp.pad(row, (0, pad)).reshape(IDX_ROWS, 128)
    cidx_g = jnp.pad(col, (0, pad)).reshape(IDX_ROWS, 128)
    cidx_s = jnp.pad(col, (0, pad), constant_values=N).reshape(IDX_ROWS, 128)
    ea_pad = jnp.pad(edge_attr, ((0, pad), (0, 0)))
    batch3 = batch.reshape(N // NBLK, 1, NBLK)
    be1_ = be1.reshape(1, -1)
    be2_ = be2.reshape(1, -1)
    bn1a_ = bn1a.reshape(1, -1)
    bn1b_ = bn1b.reshape(1, -1)
    bn2a_ = bn2a.reshape(1, -1)
    bn2b_ = bn2b.reshape(1, -1)
    bg1_ = bg1.reshape(1, -1)
    bg2_ = bg2.reshape(1, -1)

    trow, tcol = _tc1(x, batch3, u, We1, be1_, Wn1a, bn1a_)
    grow, gcol = _sc_gather(trow, tcol, ridx, cidx_g)
    en_pad, m_pad = _tc2(grow, gcol, ea_pad, We1, We2, be2_, Wn1a, Wn1b, bn1b_)
    parts = _sc_scatter(m_pad, cidx_s)
    x_new, u_new = _tc3(x, parts, batch3, u,
                        Wn2a, bn2a_, Wn2b, bn2b_, Wg1, bg1_, Wg2, bg2_)
    edge_new = en_pad[:E]
    return (x_new, edge_new, u_new)


# speedup vs baseline: 5.0394x; 1.1417x over previous
import functools

import jax
import jax.numpy as jnp
from jax import lax
from jax.experimental import pallas as pl
from jax.experimental.pallas import tpu as pltpu
from jax.experimental.pallas import tpu_sc as plsc

N = 10000
E = 320000
G = 16
D_NODE = 128
D_EDGE = 16
D_U = 16
MSG = 32

NC = 2
NS = 16
NW = NC * NS
ROWS_PER_W = 80
IDX_ROWS = NW * ROWS_PER_W
E_PAD = IDX_ROWS * 128
CH = 4
N_SUP = ROWS_PER_W // (2 * CH)
N_ACC = N + 16
ACC_W = 48
ROWS_PER_TILE = N_ACC // NS
NBLK = 1000
EBLK = 2000

_f32 = jnp.float32


def _tc1_body(xb, bb, u, We1, be1, Wn1a, bn1a, trow, tcol):
    xv = xb[...]
    b = bb[0, 0, :]
    oh = (b[:, None] == lax.broadcasted_iota(jnp.int32, (NBLK, G), 1)).astype(_f32)
    w = We1[...]
    ub1 = jnp.dot(u[...], w[2 * D_NODE + D_EDGE:, :], preferred_element_type=_f32)
    rowp = (jnp.dot(xv, w[:D_NODE, :], preferred_element_type=_f32)
            + jnp.dot(oh, ub1, preferred_element_type=_f32) + be1[0])
    xm = jnp.dot(xv, Wn1a[:D_NODE, :], preferred_element_type=_f32) + bn1a[0]
    trow[:, 0:MSG] = rowp
    trow[:, MSG:2 * MSG] = xm
    tcol[...] = jnp.dot(xv, w[D_NODE:2 * D_NODE, :], preferred_element_type=_f32)


def _tc1(x, batch3, u, We1, be1, Wn1a, bn1a):
    grid = N // NBLK
    return pl.pallas_call(
        _tc1_body,
        grid=(grid,),
        in_specs=[
            pl.BlockSpec((NBLK, D_NODE), lambda i: (i, 0)),
            pl.BlockSpec((1, 1, NBLK), lambda i: (i, 0, 0)),
            pl.BlockSpec(u.shape, lambda i: (0, 0)),
            pl.BlockSpec(We1.shape, lambda i: (0, 0)),
            pl.BlockSpec(be1.shape, lambda i: (0, 0)),
            pl.BlockSpec(Wn1a.shape, lambda i: (0, 0)),
            pl.BlockSpec(bn1a.shape, lambda i: (0, 0)),
        ],
        out_specs=[
            pl.BlockSpec((NBLK, 2 * MSG), lambda i: (i, 0)),
            pl.BlockSpec((NBLK, MSG), lambda i: (i, 0)),
        ],
        out_shape=[
            jax.ShapeDtypeStruct((N, 2 * MSG), _f32),
            jax.ShapeDtypeStruct((N, MSG), _f32),
        ],
    )(x, batch3, u, We1, be1, Wn1a, bn1a)


def _sc_gather_body(trow, tcol, ridx, cidx, grow, gcol,
                    idxr, idxc, bufr0, bufr1, bufc0, bufc1,
                    semi, semg0, semg1, semw0, semw1):
    c = lax.axis_index("c")
    s = lax.axis_index("s")
    wid = s * NC + c
    rbase = wid * ROWS_PER_W
    ebase = rbase * 128
    cpi1 = pltpu.async_copy(ridx.at[pl.ds(rbase, ROWS_PER_W)], idxr, semi)
    cpi2 = pltpu.async_copy(cidx.at[pl.ds(rbase, ROWS_PER_W)], idxc, semi)
    cpi1.wait()
    cpi2.wait()

    def g_list(k, br, bc, sem):
        r0 = k * CH
        out = []
        for j in range(CH):
            out.append((trow.at[idxr.at[r0 + j]],
                        br.at[pl.ds(j * 128, 128)], sem))
            out.append((tcol.at[idxc.at[r0 + j]],
                        bc.at[pl.ds(j * 128, 128)], sem))
        return out

    def w_list(k, br, bc, sem):
        e0 = ebase + k * CH * 128
        return [(br, grow.at[pl.ds(e0, CH * 128)], sem),
                (bc, gcol.at[pl.ds(e0, CH * 128)], sem)]

    def fire(tl):
        for a, b, m in tl:
            pltpu.async_copy(a, b, m)

    def drain(tl):
        for a, b, m in tl:
            pltpu.make_async_copy(a, b, m).wait()

    fire(g_list(0, bufr0, bufc0, semg0))

    def sup(i, carry):
        k0 = 2 * i
        k1 = 2 * i + 1

        @pl.when(i > 0)
        def _():
            drain(w_list(k1, bufr1, bufc1, semw1))

        fire(g_list(k1, bufr1, bufc1, semg1))
        drain(g_list(k0, bufr0, bufc0, semg0))
        fire(w_list(k0, bufr0, bufc0, semw0))
        drain(g_list(k1, bufr1, bufc1, semg1))
        fire(w_list(k1, bufr1, bufc1, semw1))

        @pl.when(i < N_SUP - 1)
        def _():
            drain(w_list(k0, bufr0, bufc0, semw0))
            fire(g_list(k0 + 2, bufr0, bufc0, semg0))

        return carry

    lax.fori_loop(0, N_SUP, sup, 0)
    drain(w_list(0, bufr0, bufc0, semw0))
    drain(w_list(0, bufr1, bufc1, semw1))


def _sc_gather(trow, tcol, ridx, cidx):
    mesh = plsc.VectorSubcoreMesh(core_axis_name="c", subcore_axis_name="s",
                                  num_cores=NC, num_subcores=NS)
    fn = pl.kernel(
        _sc_gather_body,
        out_type=[
            jax.ShapeDtypeStruct((E_PAD, 2 * MSG), _f32),
            jax.ShapeDtypeStruct((E_PAD, MSG), _f32),
        ],
        mesh=mesh,
        compiler_params=pltpu.CompilerParams(use_tc_tiling_on_sc=False),
        scratch_types=[
            pltpu.VMEM((ROWS_PER_W, 128), jnp.int32),
            pltpu.VMEM((ROWS_PER_W, 128), jnp.int32),
            pltpu.VMEM((CH * 128, 2 * MSG), _f32),
            pltpu.VMEM((CH * 128, 2 * MSG), _f32),
            pltpu.VMEM((CH * 128, MSG), _f32),
            pltpu.VMEM((CH * 128, MSG), _f32),
            pltpu.SemaphoreType.DMA,
            pltpu.SemaphoreType.DMA,
            pltpu.SemaphoreType.DMA,
            pltpu.SemaphoreType.DMA,
            pltpu.SemaphoreType.DMA,
        ],
    )
    return fn(trow, tcol, ridx, cidx)


def _tc2_body(growb, gcolb, eab, We1, We2, be2, Wn1a, Wn1b, bn1b, en_out, m_out):
    gr = growb[...]
    eap = jnp.dot(eab[...], We1[2 * D_NODE:2 * D_NODE + D_EDGE, :],
                  preferred_element_type=_f32)
    h1 = jnp.maximum(gr[:, 0:MSG] + gcolb[...] + eap, 0.0)
    en = jnp.dot(h1, We2[...], preferred_element_type=_f32) + be2[0]
    en_out[...] = en
    mh = jnp.maximum(
        gr[:, MSG:2 * MSG]
        + jnp.dot(en, Wn1a[D_NODE:D_NODE + D_EDGE, :], preferred_element_type=_f32),
        0.0)
    m_out[...] = jnp.dot(mh, Wn1b[...], preferred_element_type=_f32) + bn1b[0]


def _tc2(grow, gcol, ea, We1, We2, be2, Wn1a, Wn1b, bn1b):
    grid = E // EBLK
    return pl.pallas_call(
        _tc2_body,
        grid=(grid,),
        in_specs=[
            pl.BlockSpec((EBLK, 2 * MSG), lambda i: (i, 0)),
            pl.BlockSpec((EBLK, MSG), lambda i: (i, 0)),
            pl.BlockSpec((EBLK, D_EDGE), lambda i: (i, 0)),
            pl.BlockSpec(We1.shape, lambda i: (0, 0)),
            pl.BlockSpec(We2.shape, lambda i: (0, 0)),
            pl.BlockSpec(be2.shape, lambda i: (0, 0)),
            pl.BlockSpec(Wn1a.shape, lambda i: (0, 0)),
            pl.BlockSpec(Wn1b.shape, lambda i: (0, 0)),
            pl.BlockSpec(bn1b.shape, lambda i: (0, 0)),
        ],
        out_specs=[
            pl.BlockSpec((EBLK, D_EDGE), lambda i: (i, 0)),
            pl.BlockSpec((EBLK, MSG), lambda i: (i, 0)),
        ],
        out_shape=[
            jax.ShapeDtypeStruct((E, D_EDGE), _f32),
            jax.ShapeDtypeStruct((E_PAD, MSG), _f32),
        ],
    )(grow, gcol, ea, We1, We2, be2, Wn1a, Wn1b, bn1b)


def _sc_scatter_body(m_hbm, cidx, out_hbm, acc, mbuf0, mbuf1, idxc, zbuf,
                     semz, semi, seml0, seml1, sems0, sems1, semo):
    c = lax.axis_index("c")
    s = lax.axis_index("s")
    wid = s * NC + c
    rbase = wid * ROWS_PER_W
    ebase = rbase * 128
    cpi = pltpu.async_copy(cidx.at[pl.ds(rbase, ROWS_PER_W)], idxc, semi)

    def zfill(i, carry):
        zbuf[i, pl.ds(0, 16)] = jnp.zeros((16,), _f32)
        zbuf[i, pl.ds(16, 16)] = jnp.zeros((16,), _f32)
        zbuf[i, pl.ds(32, 16)] = jnp.zeros((16,), _f32)
        return carry

    lax.fori_loop(0, ROWS_PER_TILE // 2, zfill, 0)
    row0 = s * ROWS_PER_TILE
    pltpu.async_copy(zbuf, acc.at[pl.ds(row0, ROWS_PER_TILE // 2)], semz).wait()
    pltpu.async_copy(
        zbuf, acc.at[pl.ds(row0 + ROWS_PER_TILE // 2, ROWS_PER_TILE // 2)],
        semz).wait()

    def ofill(i, carry):
        mbuf0[i, pl.ds(MSG, 16)] = jnp.full((16,), 1.0, _f32)
        mbuf1[i, pl.ds(MSG, 16)] = jnp.full((16,), 1.0, _f32)
        return carry

    lax.fori_loop(0, CH * 128, ofill, 0)
    cpi.wait()
    plsc.subcore_barrier()

    def l_list(k, mb, sem):
        e0 = ebase + k * CH * 128
        return [(m_hbm.at[pl.ds(e0, CH * 128)], mb.at[:, pl.ds(0, MSG)], sem)]

    def s_list(k, mb, sem):
        r0 = k * CH
        return [(mb.at[pl.ds(j * 128, 128)], acc.at[idxc.at[r0 + j]], sem)
                for j in range(CH)]

    def fire(tl, add=False):
        for a, b, m in tl:
            pltpu.async_copy(a, b, m, add=add)

    def drain(tl):
        for a, b, m in tl:
            pltpu.make_async_copy(a, b, m).wait()

    fire(l_list(0, mbuf0, seml0))

    def sup(i, carry):
        k0 = 2 * i
        k1 = 2 * i + 1

        @pl.when(i > 0)
        def _():
            drain(s_list(k1, mbuf1, sems1))

        fire(l_list(k1, mbuf1, seml1))
        drain(l_list(k0, mbuf0, seml0))
        fire(s_list(k0, mbuf0, sems0), add=True)
        drain(l_list(k1, mbuf1, seml1))
        fire(s_list(k1, mbuf1, sems1), add=True)

        @pl.when(i < N_SUP - 1)
        def _():
            drain(s_list(k0, mbuf0, sems0))
            fire(l_list(k0 + 2, mbuf0, seml0))

        return carry

    lax.fori_loop(0, N_SUP, sup, 0)
    drain(s_list(0, mbuf0, sems0))
    drain(s_list(0, mbuf1, sems1))
    plsc.subcore_barrier()
    pltpu.async_copy(acc.at[pl.ds(row0, ROWS_PER_TILE)],
                     out_hbm.at[c, pl.ds(row0, ROWS_PER_TILE)], semo).wait()


def _sc_scatter(m_pad, cidx_s):
    mesh = plsc.VectorSubcoreMesh(core_axis_name="c", subcore_axis_name="s",
                                  num_cores=NC, num_subcores=NS)
    fn = pl.kernel(
        _sc_scatter_body,
        out_type=jax.ShapeDtypeStruct((NC, N_ACC, ACC_W), _f32),
        mesh=mesh,
        compiler_params=pltpu.CompilerParams(use_tc_tiling_on_sc=False),
        scratch_types=[
            pltpu.VMEM_SHARED((N_ACC, ACC_W), _f32),
            pltpu.VMEM((CH * 128, ACC_W), _f32),
            pltpu.VMEM((CH * 128, ACC_W), _f32),
            pltpu.VMEM((ROWS_PER_W, 128), jnp.int32),
            pltpu.VMEM((ROWS_PER_TILE // 2, ACC_W), _f32),
            pltpu.SemaphoreType.DMA,
            pltpu.SemaphoreType.DMA,
            pltpu.SemaphoreType.DMA,
            pltpu.SemaphoreType.DMA,
            pltpu.SemaphoreType.DMA,
            pltpu.SemaphoreType.DMA,
            pltpu.SemaphoreType.DMA,
        ],
    )
    return fn(m_pad, cidx_s)


def _tc3_body(xb, pb, bb, u, Wn2a, bn2a, Wn2b, bn2b, Wg1, bg1, Wg2, bg2,
              xn_out, un_out, xsum, nct):
    i = pl.program_id(0)
    p = pb[...]
    ssum = p[0] + p[1]
    agg = ssum[:, 0:MSG] / jnp.maximum(ssum[:, MSG:MSG + 1], 1.0)
    b = bb[0, 0, :]
    oh = (b[:, None] == lax.broadcasted_iota(jnp.int32, (NBLK, G), 1)).astype(_f32)
    wa = Wn2a[...]
    ub2 = jnp.dot(u[...], wa[D_NODE + MSG:, :], preferred_element_type=_f32)
    h = jnp.maximum(
        jnp.dot(xb[...], wa[:D_NODE, :], preferred_element_type=_f32)
        + jnp.dot(agg, wa[D_NODE:D_NODE + MSG, :], preferred_element_type=_f32)
        + jnp.dot(oh, ub2, preferred_element_type=_f32) + bn2a[0],
        0.0)
    xn = jnp.dot(h, Wn2b[...], preferred_element_type=_f32) + bn2b[0]
    xn_out[...] = xn

    @pl.when(i == 0)
    def _init():
        xsum[...] = jnp.zeros((G, D_NODE), _f32)
        nct[...] = jnp.zeros((G, D_NODE), _f32)

    dn = (((0,), (0,)), ((), ()))
    xsum[...] += lax.dot_general(oh, xn, dn, preferred_element_type=_f32)
    nct[...] += lax.dot_general(oh, jnp.ones((NBLK, D_NODE), _f32), dn,
                                preferred_element_type=_f32)

    @pl.when(i == (N // NBLK) - 1)
    def _fin():
        xmean = xsum[...] / jnp.maximum(nct[...], 1.0)
        uu = u[...]
        gh = jnp.maximum(
            jnp.dot(uu, Wg1[:D_U, :], preferred_element_type=_f32)
            + jnp.dot(xmean, Wg1[D_U:, :], preferred_element_type=_f32)
            + bg1[0],
            0.0)
        un_out[...] = jnp.dot(gh, Wg2[...], preferred_element_type=_f32) + bg2[0]


def _tc3(x, parts, batch3, u, Wn2a, bn2a, Wn2b, bn2b, Wg1, bg1, Wg2, bg2):
    grid = N // NBLK
    return pl.pallas_call(
        _tc3_body,
        grid=(grid,),
        in_specs=[
            pl.BlockSpec((NBLK, D_NODE), lambda i: (i, 0)),
            pl.BlockSpec((NC, NBLK, ACC_W), lambda i: (0, i, 0)),
            pl.BlockSpec((1, 1, NBLK), lambda i: (i, 0, 0)),
            pl.BlockSpec(u.shape, lambda i: (0, 0)),
            pl.BlockSpec(Wn2a.shape, lambda i: (0, 0)),
            pl.BlockSpec(bn2a.shape, lambda i: (0, 0)),
            pl.BlockSpec(Wn2b.shape, lambda i: (0, 0)),
            pl.BlockSpec(bn2b.shape, lambda i: (0, 0)),
            pl.BlockSpec(Wg1.shape, lambda i: (0, 0)),
            pl.BlockSpec(bg1.shape, lambda i: (0, 0)),
            pl.BlockSpec(Wg2.shape, lambda i: (0, 0)),
            pl.BlockSpec(bg2.shape, lambda i: (0, 0)),
        ],
        out_specs=[
            pl.BlockSpec((NBLK, D_NODE), lambda i: (i, 0)),
            pl.BlockSpec((G, D_U), lambda i: (0, 0)),
        ],
        out_shape=[
            jax.ShapeDtypeStruct((N, D_NODE), _f32),
            jax.ShapeDtypeStruct((G, D_U), _f32),
        ],
        scratch_shapes=[
            pltpu.VMEM((G, D_NODE), _f32),
            pltpu.VMEM((G, D_NODE), _f32),
        ],
    )(x, parts, batch3, u, Wn2a, bn2a, Wn2b, bn2b, Wg1, bg1, Wg2, bg2)


def kernel(x, edge_index, edge_attr, u, batch,
           We1, be1, We2, be2,
           Wn1a, bn1a, Wn1b, bn1b,
           Wn2a, bn2a, Wn2b, bn2b,
           Wg1, bg1, Wg2, bg2):
    row = edge_index[0]
    col = edge_index[1]
    pad = E_PAD - E
    ridx = jnp.pad(row, (0, pad)).reshape(IDX_ROWS, 128)
    cidx_g = jnp.pad(col, (0, pad)).reshape(IDX_ROWS, 128)
    cidx_s = jnp.pad(col, (0, pad), constant_values=N).reshape(IDX_ROWS, 128)
    batch3 = batch.reshape(N // NBLK, 1, NBLK)
    be1_ = be1.reshape(1, -1)
    be2_ = be2.reshape(1, -1)
    bn1a_ = bn1a.reshape(1, -1)
    bn1b_ = bn1b.reshape(1, -1)
    bn2a_ = bn2a.reshape(1, -1)
    bn2b_ = bn2b.reshape(1, -1)
    bg1_ = bg1.reshape(1, -1)
    bg2_ = bg2.reshape(1, -1)

    trow, tcol = _tc1(x, batch3, u, We1, be1_, Wn1a, bn1a_)
    grow, gcol = _sc_gather(trow, tcol, ridx, cidx_g)
    edge_new, m_pad = _tc2(grow, gcol, edge_attr, We1, We2, be2_,
                           Wn1a, Wn1b, bn1b_)
    parts = _sc_scatter(m_pad, cidx_s)
    x_new, u_new = _tc3(x, parts, batch3, u,
                        Wn2a, bn2a_, Wn2b, bn2b_, Wg1, bg1_, Wg2, bg2_)
    return (x_new, edge_new, u_new)


# speedup vs baseline: 7.1570x; 1.4202x over previous
import functools

import jax
import jax.numpy as jnp
from jax import lax
from jax.experimental import pallas as pl
from jax.experimental.pallas import tpu as pltpu
from jax.experimental.pallas import tpu_sc as plsc

N = 10000
E = 320000
G = 16
D_NODE = 128
D_EDGE = 16
D_U = 16
MSG = 32

NC = 2
NS = 16
NW = NC * NS
ROWS_PER_W = 80
IDX_ROWS = NW * ROWS_PER_W
E_PAD = IDX_ROWS * 128
CH = 4
N_SUP = ROWS_PER_W // (2 * CH)
N_ACC = N + 16
ACC_W = 48
ROWS_PER_TILE = N_ACC // NS
NBLK = 1000
EBLK = 3200

_f32 = jnp.float32


def _tc1_body(xb, bb, u, We1, be1, Wn1a, bn1a, trow, tcol):
    xv = xb[...]
    b = bb[0, 0, :]
    oh = (b[:, None] == lax.broadcasted_iota(jnp.int32, (NBLK, G), 1)).astype(_f32)
    w = We1[...]
    ub1 = jnp.dot(u[...], w[2 * D_NODE + D_EDGE:, :], preferred_element_type=_f32)
    rowp = (jnp.dot(xv, w[:D_NODE, :], preferred_element_type=_f32)
            + jnp.dot(oh, ub1, preferred_element_type=_f32) + be1[0])
    xm = jnp.dot(xv, Wn1a[:D_NODE, :], preferred_element_type=_f32) + bn1a[0]
    trow[:, 0:MSG] = rowp
    trow[:, MSG:2 * MSG] = xm
    tcol[...] = jnp.dot(xv, w[D_NODE:2 * D_NODE, :], preferred_element_type=_f32)


def _tc1(x, batch3, u, We1, be1, Wn1a, bn1a):
    grid = N // NBLK
    return pl.pallas_call(
        _tc1_body,
        grid=(grid,),
        in_specs=[
            pl.BlockSpec((NBLK, D_NODE), lambda i: (i, 0)),
            pl.BlockSpec((1, 1, NBLK), lambda i: (i, 0, 0)),
            pl.BlockSpec(u.shape, lambda i: (0, 0)),
            pl.BlockSpec(We1.shape, lambda i: (0, 0)),
            pl.BlockSpec(be1.shape, lambda i: (0, 0)),
            pl.BlockSpec(Wn1a.shape, lambda i: (0, 0)),
            pl.BlockSpec(bn1a.shape, lambda i: (0, 0)),
        ],
        out_specs=[
            pl.BlockSpec((NBLK, 2 * MSG), lambda i: (i, 0)),
            pl.BlockSpec((NBLK, MSG), lambda i: (i, 0)),
        ],
        out_shape=[
            jax.ShapeDtypeStruct((N, 2 * MSG), _f32),
            jax.ShapeDtypeStruct((N, MSG), _f32),
        ],
    )(x, batch3, u, We1, be1, Wn1a, bn1a)


def _sc_gather_body(trow, tcol, ridx, cidx, grow,
                    idxr, idxc, bufr0, bufr1, bufc0, bufc1,
                    semi, semg0, semg1, semw0, semw1):
    c = lax.axis_index("c")
    s = lax.axis_index("s")
    wid = s * NC + c
    rbase = wid * ROWS_PER_W
    ebase = rbase * 128
    cpi1 = pltpu.async_copy(ridx.at[pl.ds(rbase, ROWS_PER_W)], idxr, semi)
    cpi2 = pltpu.async_copy(cidx.at[pl.ds(rbase, ROWS_PER_W)], idxc, semi)
    cpi1.wait()
    cpi2.wait()

    def g_list(k, br, bc, sem):
        r0 = k * CH
        out = []
        for j in range(CH):
            out.append((trow.at[idxr.at[r0 + j]],
                        br.at[pl.ds(j * 128, 128)], sem))
            out.append((tcol.at[idxc.at[r0 + j]],
                        bc.at[pl.ds(j * 128, 128)], sem))
        return out

    def w_list(k, br, bc, sem):
        e0 = ebase + k * CH * 128
        return [(br, grow.at[pl.ds(e0, CH * 128), pl.ds(0, 2 * MSG)], sem),
                (bc, grow.at[pl.ds(e0, CH * 128), pl.ds(2 * MSG, MSG)], sem)]

    def fire(tl):
        for a, b, m in tl:
            pltpu.async_copy(a, b, m)

    def drain(tl):
        for a, b, m in tl:
            pltpu.make_async_copy(a, b, m).wait()

    fire(g_list(0, bufr0, bufc0, semg0))

    def sup(i, carry):
        k0 = 2 * i
        k1 = 2 * i + 1

        @pl.when(i > 0)
        def _():
            drain(w_list(k1, bufr1, bufc1, semw1))

        fire(g_list(k1, bufr1, bufc1, semg1))
        drain(g_list(k0, bufr0, bufc0, semg0))
        fire(w_list(k0, bufr0, bufc0, semw0))
        drain(g_list(k1, bufr1, bufc1, semg1))
        fire(w_list(k1, bufr1, bufc1, semw1))

        @pl.when(i < N_SUP - 1)
        def _():
            drain(w_list(k0, bufr0, bufc0, semw0))
            fire(g_list(k0 + 2, bufr0, bufc0, semg0))

        return carry

    lax.fori_loop(0, N_SUP, sup, 0)
    drain(w_list(0, bufr0, bufc0, semw0))
    drain(w_list(0, bufr1, bufc1, semw1))


def _sc_gather(trow, tcol, ridx, cidx):
    mesh = plsc.VectorSubcoreMesh(core_axis_name="c", subcore_axis_name="s",
                                  num_cores=NC, num_subcores=NS)
    fn = pl.kernel(
        _sc_gather_body,
        out_type=jax.ShapeDtypeStruct((E_PAD, 128), _f32),
        mesh=mesh,
        compiler_params=pltpu.CompilerParams(use_tc_tiling_on_sc=False),
        scratch_types=[
            pltpu.VMEM((ROWS_PER_W, 128), jnp.int32),
            pltpu.VMEM((ROWS_PER_W, 128), jnp.int32),
            pltpu.VMEM((CH * 128, 2 * MSG), _f32),
            pltpu.VMEM((CH * 128, 2 * MSG), _f32),
            pltpu.VMEM((CH * 128, MSG), _f32),
            pltpu.VMEM((CH * 128, MSG), _f32),
            pltpu.SemaphoreType.DMA,
            pltpu.SemaphoreType.DMA,
            pltpu.SemaphoreType.DMA,
            pltpu.SemaphoreType.DMA,
            pltpu.SemaphoreType.DMA,
        ],
    )
    return fn(trow, tcol, ridx, cidx)


def _tc2_body(gb, eab, We1, We2, be2, Wn1a, Wn1b, bn1b, en_out, m_out):
    g = gb[...]
    eap = jnp.dot(eab[...], We1[2 * D_NODE:2 * D_NODE + D_EDGE, :],
                  preferred_element_type=_f32)
    h1 = jnp.maximum(g[:, 0:MSG] + g[:, 2 * MSG:3 * MSG] + eap, 0.0)
    en = jnp.dot(h1, We2[...], preferred_element_type=_f32) + be2[0]
    en_out[...] = en
    mh = jnp.maximum(
        g[:, MSG:2 * MSG]
        + jnp.dot(en, Wn1a[D_NODE:D_NODE + D_EDGE, :], preferred_element_type=_f32),
        0.0)
    m = jnp.dot(mh, Wn1b[...], preferred_element_type=_f32) + bn1b[0]
    m_out[:, 0:MSG] = m
    m_out[:, MSG:] = jnp.zeros((EBLK, 128 - MSG), _f32)


def _tc2(g, ea, We1, We2, be2, Wn1a, Wn1b, bn1b):
    grid = E // EBLK
    return pl.pallas_call(
        _tc2_body,
        grid=(grid,),
        in_specs=[
            pl.BlockSpec((EBLK, 128), lambda i: (i, 0)),
            pl.BlockSpec((EBLK, D_EDGE), lambda i: (i, 0)),
            pl.BlockSpec(We1.shape, lambda i: (0, 0)),
            pl.BlockSpec(We2.shape, lambda i: (0, 0)),
            pl.BlockSpec(be2.shape, lambda i: (0, 0)),
            pl.BlockSpec(Wn1a.shape, lambda i: (0, 0)),
            pl.BlockSpec(Wn1b.shape, lambda i: (0, 0)),
            pl.BlockSpec(bn1b.shape, lambda i: (0, 0)),
        ],
        out_specs=[
            pl.BlockSpec((EBLK, D_EDGE), lambda i: (i, 0)),
            pl.BlockSpec((EBLK, 128), lambda i: (i, 0)),
        ],
        out_shape=[
            jax.ShapeDtypeStruct((E, D_EDGE), _f32),
            jax.ShapeDtypeStruct((E_PAD, 128), _f32),
        ],
    )(g, ea, We1, We2, be2, Wn1a, Wn1b, bn1b)


def _sc_scatter_body(m_hbm, cidx, out_hbm, acc, mbuf0, mbuf1, idxc, zbuf,
                     semz, semi, seml0, seml1, sems0, sems1, semo):
    c = lax.axis_index("c")
    s = lax.axis_index("s")
    wid = s * NC + c
    rbase = wid * ROWS_PER_W
    ebase = rbase * 128
    cpi = pltpu.async_copy(cidx.at[pl.ds(rbase, ROWS_PER_W)], idxc, semi)

    def zfill(i, carry):
        zbuf[i, pl.ds(0, 16)] = jnp.zeros((16,), _f32)
        zbuf[i, pl.ds(16, 16)] = jnp.zeros((16,), _f32)
        zbuf[i, pl.ds(32, 16)] = jnp.zeros((16,), _f32)
        return carry

    lax.fori_loop(0, ROWS_PER_TILE // 2, zfill, 0)
    row0 = s * ROWS_PER_TILE
    pltpu.async_copy(zbuf, acc.at[pl.ds(row0, ROWS_PER_TILE // 2)], semz).wait()
    pltpu.async_copy(
        zbuf, acc.at[pl.ds(row0 + ROWS_PER_TILE // 2, ROWS_PER_TILE // 2)],
        semz).wait()

    def ofill(i, carry):
        mbuf0[i, pl.ds(MSG, 16)] = jnp.full((16,), 1.0, _f32)
        mbuf1[i, pl.ds(MSG, 16)] = jnp.full((16,), 1.0, _f32)
        return carry

    lax.fori_loop(0, CH * 128, ofill, 0)
    cpi.wait()
    plsc.subcore_barrier()

    def l_list(k, mb, sem):
        e0 = ebase + k * CH * 128
        return [(m_hbm.at[pl.ds(e0, CH * 128), pl.ds(0, MSG)],
                 mb.at[:, pl.ds(0, MSG)], sem)]

    def s_list(k, mb, sem):
        r0 = k * CH
        return [(mb.at[pl.ds(j * 128, 128)], acc.at[idxc.at[r0 + j]], sem)
                for j in range(CH)]

    def fire(tl, add=False):
        for a, b, m in tl:
            pltpu.async_copy(a, b, m, add=add)

    def drain(tl):
        for a, b, m in tl:
            pltpu.make_async_copy(a, b, m).wait()

    fire(l_list(0, mbuf0, seml0))

    def sup(i, carry):
        k0 = 2 * i
        k1 = 2 * i + 1

        @pl.when(i > 0)
        def _():
            drain(s_list(k1, mbuf1, sems1))

        fire(l_list(k1, mbuf1, seml1))
        drain(l_list(k0, mbuf0, seml0))
        fire(s_list(k0, mbuf0, sems0), add=True)
        drain(l_list(k1, mbuf1, seml1))
        fire(s_list(k1, mbuf1, sems1), add=True)

        @pl.when(i < N_SUP - 1)
        def _():
            drain(s_list(k0, mbuf0, sems0))
            fire(l_list(k0 + 2, mbuf0, seml0))

        return carry

    lax.fori_loop(0, N_SUP, sup, 0)
    drain(s_list(0, mbuf0, sems0))
    drain(s_list(0, mbuf1, sems1))
    plsc.subcore_barrier()
    pltpu.async_copy(acc.at[pl.ds(row0, ROWS_PER_TILE)],
                     out_hbm.at[c, pl.ds(row0, ROWS_PER_TILE)], semo).wait()


def _sc_scatter(m_pad, cidx_s):
    mesh = plsc.VectorSubcoreMesh(core_axis_name="c", subcore_axis_name="s",
                                  num_cores=NC, num_subcores=NS)
    fn = pl.kernel(
        _sc_scatter_body,
        out_type=jax.ShapeDtypeStruct((NC, N_ACC, ACC_W), _f32),
        mesh=mesh,
        compiler_params=pltpu.CompilerParams(use_tc_tiling_on_sc=False),
        scratch_types=[
            pltpu.VMEM_SHARED((N_ACC, ACC_W), _f32),
            pltpu.VMEM((CH * 128, ACC_W), _f32),
            pltpu.VMEM((CH * 128, ACC_W), _f32),
            pltpu.VMEM((ROWS_PER_W, 128), jnp.int32),
            pltpu.VMEM((ROWS_PER_TILE // 2, ACC_W), _f32),
            pltpu.SemaphoreType.DMA,
            pltpu.SemaphoreType.DMA,
            pltpu.SemaphoreType.DMA,
            pltpu.SemaphoreType.DMA,
            pltpu.SemaphoreType.DMA,
            pltpu.SemaphoreType.DMA,
            pltpu.SemaphoreType.DMA,
        ],
    )
    return fn(m_pad, cidx_s)


def _tc3_body(xb, pb, bb, u, Wn2a, bn2a, Wn2b, bn2b, Wg1, bg1, Wg2, bg2,
              xn_out, un_out, xsum, nct):
    i = pl.program_id(0)
    p = pb[...]
    ssum = p[0] + p[1]
    agg = ssum[:, 0:MSG] / jnp.maximum(ssum[:, MSG:MSG + 1], 1.0)
    b = bb[0, 0, :]
    oh = (b[:, None] == lax.broadcasted_iota(jnp.int32, (NBLK, G), 1)).astype(_f32)
    wa = Wn2a[...]
    ub2 = jnp.dot(u[...], wa[D_NODE + MSG:, :], preferred_element_type=_f32)
    h = jnp.maximum(
        jnp.dot(xb[...], wa[:D_NODE, :], preferred_element_type=_f32)
        + jnp.dot(agg, wa[D_NODE:D_NODE + MSG, :], preferred_element_type=_f32)
        + jnp.dot(oh, ub2, preferred_element_type=_f32) + bn2a[0],
        0.0)
    xn = jnp.dot(h, Wn2b[...], preferred_element_type=_f32) + bn2b[0]
    xn_out[...] = xn

    @pl.when(i == 0)
    def _init():
        xsum[...] = jnp.zeros((G, D_NODE), _f32)
        nct[...] = jnp.zeros((G, D_NODE), _f32)

    dn = (((0,), (0,)), ((), ()))
    xsum[...] += lax.dot_general(oh, xn, dn, preferred_element_type=_f32)
    nct[...] += lax.dot_general(oh, jnp.ones((NBLK, D_NODE), _f32), dn,
                                preferred_element_type=_f32)

    @pl.when(i == (N // NBLK) - 1)
    def _fin():
        xmean = xsum[...] / jnp.maximum(nct[...], 1.0)
        uu = u[...]
        gh = jnp.maximum(
            jnp.dot(uu, Wg1[:D_U, :], preferred_element_type=_f32)
            + jnp.dot(xmean, Wg1[D_U:, :], preferred_element_type=_f32)
            + bg1[0],
            0.0)
        un_out[...] = jnp.dot(gh, Wg2[...], preferred_element_type=_f32) + bg2[0]


def _tc3(x, parts, batch3, u, Wn2a, bn2a, Wn2b, bn2b, Wg1, bg1, Wg2, bg2):
    grid = N // NBLK
    return pl.pallas_call(
        _tc3_body,
        grid=(grid,),
        in_specs=[
            pl.BlockSpec((NBLK, D_NODE), lambda i: (i, 0)),
            pl.BlockSpec((NC, NBLK, ACC_W), lambda i: (0, i, 0)),
            pl.BlockSpec((1, 1, NBLK), lambda i: (i, 0, 0)),
            pl.BlockSpec(u.shape, lambda i: (0, 0)),
            pl.BlockSpec(Wn2a.shape, lambda i: (0, 0)),
            pl.BlockSpec(bn2a.shape, lambda i: (0, 0)),
            pl.BlockSpec(Wn2b.shape, lambda i: (0, 0)),
            pl.BlockSpec(bn2b.shape, lambda i: (0, 0)),
            pl.BlockSpec(Wg1.shape, lambda i: (0, 0)),
            pl.BlockSpec(bg1.shape, lambda i: (0, 0)),
            pl.BlockSpec(Wg2.shape, lambda i: (0, 0)),
            pl.BlockSpec(bg2.shape, lambda i: (0, 0)),
        ],
        out_specs=[
            pl.BlockSpec((NBLK, D_NODE), lambda i: (i, 0)),
            pl.BlockSpec((G, D_U), lambda i: (0, 0)),
        ],
        out_shape=[
            jax.ShapeDtypeStruct((N, D_NODE), _f32),
            jax.ShapeDtypeStruct((G, D_U), _f32),
        ],
        scratch_shapes=[
            pltpu.VMEM((G, D_NODE), _f32),
            pltpu.VMEM((G, D_NODE), _f32),
        ],
    )(x, parts, batch3, u, Wn2a, bn2a, Wn2b, bn2b, Wg1, bg1, Wg2, bg2)


def kernel(x, edge_index, edge_attr, u, batch,
           We1, be1, We2, be2,
           Wn1a, bn1a, Wn1b, bn1b,
           Wn2a, bn2a, Wn2b, bn2b,
           Wg1, bg1, Wg2, bg2):
    row = edge_index[0]
    col = edge_index[1]
    pad = E_PAD - E
    ridx = jnp.pad(row, (0, pad)).reshape(IDX_ROWS, 128)
    cidx_g = jnp.pad(col, (0, pad)).reshape(IDX_ROWS, 128)
    cidx_s = jnp.pad(col, (0, pad), constant_values=N).reshape(IDX_ROWS, 128)
    batch3 = batch.reshape(N // NBLK, 1, NBLK)
    be1_ = be1.reshape(1, -1)
    be2_ = be2.reshape(1, -1)
    bn1a_ = bn1a.reshape(1, -1)
    bn1b_ = bn1b.reshape(1, -1)
    bn2a_ = bn2a.reshape(1, -1)
    bn2b_ = bn2b.reshape(1, -1)
    bg1_ = bg1.reshape(1, -1)
    bg2_ = bg2.reshape(1, -1)

    trow, tcol = _tc1(x, batch3, u, We1, be1_, Wn1a, bn1a_)
    g = _sc_gather(trow, tcol, ridx, cidx_g)
    edge_new, m128 = _tc2(g, edge_attr, We1, We2, be2_, Wn1a, Wn1b, bn1b_)
    parts = _sc_scatter(m128, cidx_s)
    x_new, u_new = _tc3(x, parts, batch3, u,
                        Wn2a, bn2a_, Wn2b, bn2b_, Wg1, bg1_, Wg2, bg2_)
    return (x_new, edge_new, u_new)


# speedup vs baseline: 7.7037x; 1.0764x over previous
import functools

import jax
import jax.numpy as jnp
from jax import lax
from jax.experimental import pallas as pl
from jax.experimental.pallas import tpu as pltpu
from jax.experimental.pallas import tpu_sc as plsc

N = 10000
E = 320000
G = 16
D_NODE = 128
D_EDGE = 16
D_U = 16
MSG = 32

NC = 2
NS = 16
NW = NC * NS
EH = E // 2
ROWS_PER_W = 40
IDX_ROWS_H = NW * ROWS_PER_W
E_PAD_H = IDX_ROWS_H * 128
CH = 4
N_SUP = ROWS_PER_W // (2 * CH)
N_ACC = N + 16
ACC_W = 48
ROWS_PER_TILE = N_ACC // NS
NBLK = 1000
EBLK = 3200

_f32 = jnp.float32


def _tc1_body(xb, bb, u, We1, be1, Wn1a, bn1a, trow, tcol):
    xv = xb[...]
    b = bb[0, 0, :]
    oh = (b[:, None] == lax.broadcasted_iota(jnp.int32, (NBLK, G), 1)).astype(_f32)
    w = We1[...]
    ub1 = jnp.dot(u[...], w[2 * D_NODE + D_EDGE:, :], preferred_element_type=_f32)
    rowp = (jnp.dot(xv, w[:D_NODE, :], preferred_element_type=_f32)
            + jnp.dot(oh, ub1, preferred_element_type=_f32) + be1[0])
    xm = jnp.dot(xv, Wn1a[:D_NODE, :], preferred_element_type=_f32) + bn1a[0]
    trow[:, 0:MSG] = rowp
    trow[:, MSG:2 * MSG] = xm
    tcol[...] = jnp.dot(xv, w[D_NODE:2 * D_NODE, :], preferred_element_type=_f32)


def _tc1(x, batch3, u, We1, be1, Wn1a, bn1a):
    grid = N // NBLK
    return pl.pallas_call(
        _tc1_body,
        grid=(grid,),
        in_specs=[
            pl.BlockSpec((NBLK, D_NODE), lambda i: (i, 0)),
            pl.BlockSpec((1, 1, NBLK), lambda i: (i, 0, 0)),
            pl.BlockSpec(u.shape, lambda i: (0, 0)),
            pl.BlockSpec(We1.shape, lambda i: (0, 0)),
            pl.BlockSpec(be1.shape, lambda i: (0, 0)),
            pl.BlockSpec(Wn1a.shape, lambda i: (0, 0)),
            pl.BlockSpec(bn1a.shape, lambda i: (0, 0)),
        ],
        out_specs=[
            pl.BlockSpec((NBLK, 2 * MSG), lambda i: (i, 0)),
            pl.BlockSpec((NBLK, MSG), lambda i: (i, 0)),
        ],
        out_shape=[
            jax.ShapeDtypeStruct((N, 2 * MSG), _f32),
            jax.ShapeDtypeStruct((N, MSG), _f32),
        ],
    )(x, batch3, u, We1, be1, Wn1a, bn1a)


def _sc_gather_body(trow, tcol, ridx, cidx, grow,
                    idxr, idxc, bufr0, bufr1, bufc0, bufc1,
                    semi, semg0, semg1, semw0, semw1):
    c = lax.axis_index("c")
    s = lax.axis_index("s")
    wid = s * NC + c
    rbase = wid * ROWS_PER_W
    ebase = rbase * 128
    cpi1 = pltpu.async_copy(ridx.at[pl.ds(rbase, ROWS_PER_W)], idxr, semi)
    cpi2 = pltpu.async_copy(cidx.at[pl.ds(rbase, ROWS_PER_W)], idxc, semi)
    cpi1.wait()
    cpi2.wait()

    def g_list(k, br, bc, sem):
        r0 = k * CH
        out = []
        for j in range(CH):
            out.append((trow.at[idxr.at[r0 + j]],
                        br.at[pl.ds(j * 128, 128)], sem))
            out.append((tcol.at[idxc.at[r0 + j]],
                        bc.at[pl.ds(j * 128, 128)], sem))
        return out

    def w_list(k, br, bc, sem):
        e0 = ebase + k * CH * 128
        return [(br, grow.at[pl.ds(e0, CH * 128), pl.ds(0, 2 * MSG)], sem),
                (bc, grow.at[pl.ds(e0, CH * 128), pl.ds(2 * MSG, MSG)], sem)]

    def fire(tl):
        for a, b, m in tl:
            pltpu.async_copy(a, b, m)

    def drain(tl):
        for a, b, m in tl:
            pltpu.make_async_copy(a, b, m).wait()

    fire(g_list(0, bufr0, bufc0, semg0))

    def sup(i, carry):
        k0 = 2 * i
        k1 = 2 * i + 1

        @pl.when(i > 0)
        def _():
            drain(w_list(k1, bufr1, bufc1, semw1))

        fire(g_list(k1, bufr1, bufc1, semg1))
        drain(g_list(k0, bufr0, bufc0, semg0))
        fire(w_list(k0, bufr0, bufc0, semw0))
        drain(g_list(k1, bufr1, bufc1, semg1))
        fire(w_list(k1, bufr1, bufc1, semw1))

        @pl.when(i < N_SUP - 1)
        def _():
            drain(w_list(k0, bufr0, bufc0, semw0))
            fire(g_list(k0 + 2, bufr0, bufc0, semg0))

        return carry

    lax.fori_loop(0, N_SUP, sup, 0)
    drain(w_list(0, bufr0, bufc0, semw0))
    drain(w_list(0, bufr1, bufc1, semw1))


def _sc_gather(trow, tcol, ridx, cidx):
    mesh = plsc.VectorSubcoreMesh(core_axis_name="c", subcore_axis_name="s",
                                  num_cores=NC, num_subcores=NS)
    fn = pl.kernel(
        _sc_gather_body,
        out_type=jax.ShapeDtypeStruct((E_PAD_H, 128), _f32),
        mesh=mesh,
        compiler_params=pltpu.CompilerParams(use_tc_tiling_on_sc=False),
        scratch_types=[
            pltpu.VMEM((ROWS_PER_W, 128), jnp.int32),
            pltpu.VMEM((ROWS_PER_W, 128), jnp.int32),
            pltpu.VMEM((CH * 128, 2 * MSG), _f32),
            pltpu.VMEM((CH * 128, 2 * MSG), _f32),
            pltpu.VMEM((CH * 128, MSG), _f32),
            pltpu.VMEM((CH * 128, MSG), _f32),
            pltpu.SemaphoreType.DMA,
            pltpu.SemaphoreType.DMA,
            pltpu.SemaphoreType.DMA,
            pltpu.SemaphoreType.DMA,
            pltpu.SemaphoreType.DMA,
        ],
    )
    return fn(trow, tcol, ridx, cidx)


def _tc2_body(gb, eab, We1, We2, be2, Wn1a, Wn1b, bn1b, *rest):
    en_out, m_out = rest[-2], rest[-1]
    g = gb[...]
    eap = jnp.dot(eab[...], We1[2 * D_NODE:2 * D_NODE + D_EDGE, :],
                  preferred_element_type=_f32)
    h1 = jnp.maximum(g[:, 0:MSG] + g[:, 2 * MSG:3 * MSG] + eap, 0.0)
    en = jnp.dot(h1, We2[...], preferred_element_type=_f32) + be2[0]
    en_out[...] = en
    mh = jnp.maximum(
        g[:, MSG:2 * MSG]
        + jnp.dot(en, Wn1a[D_NODE:D_NODE + D_EDGE, :], preferred_element_type=_f32),
        0.0)
    m = jnp.dot(mh, Wn1b[...], preferred_element_type=_f32) + bn1b[0]
    m_out[:, 0:MSG] = m
    m_out[:, MSG:] = jnp.zeros((EBLK, 128 - MSG), _f32)


def _tc2(g, ea, We1, We2, be2, Wn1a, Wn1b, bn1b, half, en_prev=None):
    grid = EH // EBLK
    off = half * grid
    in_specs = [
        pl.BlockSpec((EBLK, 128), lambda i: (i, 0)),
        pl.BlockSpec((EBLK, D_EDGE), lambda i: (i + off, 0)),
        pl.BlockSpec(We1.shape, lambda i: (0, 0)),
        pl.BlockSpec(We2.shape, lambda i: (0, 0)),
        pl.BlockSpec(be2.shape, lambda i: (0, 0)),
        pl.BlockSpec(Wn1a.shape, lambda i: (0, 0)),
        pl.BlockSpec(Wn1b.shape, lambda i: (0, 0)),
        pl.BlockSpec(bn1b.shape, lambda i: (0, 0)),
    ]
    args = [g, ea, We1, We2, be2, Wn1a, Wn1b, bn1b]
    aliases = {}
    if en_prev is not None:
        in_specs.append(pl.BlockSpec(memory_space=pl.ANY))
        args.append(en_prev)
        aliases = {8: 0}
    return pl.pallas_call(
        _tc2_body,
        grid=(grid,),
        in_specs=in_specs,
        out_specs=[
            pl.BlockSpec((EBLK, D_EDGE), lambda i: (i + off, 0)),
            pl.BlockSpec((EBLK, 128), lambda i: (i, 0)),
        ],
        out_shape=[
            jax.ShapeDtypeStruct((E, D_EDGE), _f32),
            jax.ShapeDtypeStruct((E_PAD_H, 128), _f32),
        ],
        input_output_aliases=aliases,
    )(*args)


def _sc_scatter_body(m_a, m_b, ca, cb, out_hbm, acc, mbuf0, mbuf1, idxc, zbuf,
                     semz, semi, seml0, seml1, sems0, sems1, semo):
    c = lax.axis_index("c")
    s = lax.axis_index("s")
    wid = s * NC + c
    rbase = wid * ROWS_PER_W
    ebase = rbase * 128

    def zfill(i, carry):
        zbuf[i, pl.ds(0, 16)] = jnp.zeros((16,), _f32)
        zbuf[i, pl.ds(16, 16)] = jnp.zeros((16,), _f32)
        zbuf[i, pl.ds(32, 16)] = jnp.zeros((16,), _f32)
        return carry

    lax.fori_loop(0, ROWS_PER_TILE // 2, zfill, 0)
    row0 = s * ROWS_PER_TILE
    pltpu.async_copy(zbuf, acc.at[pl.ds(row0, ROWS_PER_TILE // 2)], semz).wait()
    pltpu.async_copy(
        zbuf, acc.at[pl.ds(row0 + ROWS_PER_TILE // 2, ROWS_PER_TILE // 2)],
        semz).wait()

    def ofill(i, carry):
        mbuf0[i, pl.ds(MSG, 16)] = jnp.full((16,), 1.0, _f32)
        mbuf1[i, pl.ds(MSG, 16)] = jnp.full((16,), 1.0, _f32)
        return carry

    lax.fori_loop(0, CH * 128, ofill, 0)
    plsc.subcore_barrier()

    def fire(tl, add=False):
        for a, b, m in tl:
            pltpu.async_copy(a, b, m, add=add)

    def drain(tl):
        for a, b, m in tl:
            pltpu.make_async_copy(a, b, m).wait()

    def run_half(m_hbm, cidx):
        pltpu.async_copy(cidx.at[pl.ds(rbase, ROWS_PER_W)], idxc, semi).wait()

        def l_list(k, mb, sem):
            e0 = ebase + k * CH * 128
            return [(m_hbm.at[pl.ds(e0, CH * 128), pl.ds(0, MSG)],
                     mb.at[:, pl.ds(0, MSG)], sem)]

        def s_list(k, mb, sem):
            r0 = k * CH
            return [(mb.at[pl.ds(j * 128, 128)], acc.at[idxc.at[r0 + j]], sem)
                    for j in range(CH)]

        fire(l_list(0, mbuf0, seml0))

        def sup(i, carry):
            k0 = 2 * i
            k1 = 2 * i + 1

            @pl.when(i > 0)
            def _():
                drain(s_list(k1, mbuf1, sems1))

            fire(l_list(k1, mbuf1, seml1))
            drain(l_list(k0, mbuf0, seml0))
            fire(s_list(k0, mbuf0, sems0), add=True)
            drain(l_list(k1, mbuf1, seml1))
            fire(s_list(k1, mbuf1, sems1), add=True)

            @pl.when(i < N_SUP - 1)
            def _():
                drain(s_list(k0, mbuf0, sems0))
                fire(l_list(k0 + 2, mbuf0, seml0))

            return carry

        lax.fori_loop(0, N_SUP, sup, 0)
        drain(s_list(0, mbuf0, sems0))
        drain(s_list(0, mbuf1, sems1))

    run_half(m_a, ca)
    run_half(m_b, cb)
    plsc.subcore_barrier()
    pltpu.async_copy(acc.at[pl.ds(row0, ROWS_PER_TILE)],
                     out_hbm.at[c, pl.ds(row0, ROWS_PER_TILE)], semo).wait()


def _sc_scatter(m_a, m_b, ca, cb):
    mesh = plsc.VectorSubcoreMesh(core_axis_name="c", subcore_axis_name="s",
                                  num_cores=NC, num_subcores=NS)
    fn = pl.kernel(
        _sc_scatter_body,
        out_type=jax.ShapeDtypeStruct((NC, N_ACC, ACC_W), _f32),
        mesh=mesh,
        compiler_params=pltpu.CompilerParams(use_tc_tiling_on_sc=False),
        scratch_types=[
            pltpu.VMEM_SHARED((N_ACC, ACC_W), _f32),
            pltpu.VMEM((CH * 128, ACC_W), _f32),
            pltpu.VMEM((CH * 128, ACC_W), _f32),
            pltpu.VMEM((ROWS_PER_W, 128), jnp.int32),
            pltpu.VMEM((ROWS_PER_TILE // 2, ACC_W), _f32),
            pltpu.SemaphoreType.DMA,
            pltpu.SemaphoreType.DMA,
            pltpu.SemaphoreType.DMA,
            pltpu.SemaphoreType.DMA,
            pltpu.SemaphoreType.DMA,
            pltpu.SemaphoreType.DMA,
            pltpu.SemaphoreType.DMA,
        ],
    )
    return fn(m_a, m_b, ca, cb)


def _tc3_body(xb, pb, bb, u, Wn2a, bn2a, Wn2b, bn2b, Wg1, bg1, Wg2, bg2,
              xn_out, un_out, xsum, nct):
    i = pl.program_id(0)
    p = pb[...]
    ssum = p[0] + p[1]
    agg = ssum[:, 0:MSG] / jnp.maximum(ssum[:, MSG:MSG + 1], 1.0)
    b = bb[0, 0, :]
    oh = (b[:, None] == lax.broadcasted_iota(jnp.int32, (NBLK, G), 1)).astype(_f32)
    wa = Wn2a[...]
    ub2 = jnp.dot(u[...], wa[D_NODE + MSG:, :], preferred_element_type=_f32)
    h = jnp.maximum(
        jnp.dot(xb[...], wa[:D_NODE, :], preferred_element_type=_f32)
        + jnp.dot(agg, wa[D_NODE:D_NODE + MSG, :], preferred_element_type=_f32)
        + jnp.dot(oh, ub2, preferred_element_type=_f32) + bn2a[0],
        0.0)
    xn = jnp.dot(h, Wn2b[...], preferred_element_type=_f32) + bn2b[0]
    xn_out[...] = xn

    @pl.when(i == 0)
    def _init():
        xsum[...] = jnp.zeros((G, D_NODE), _f32)
        nct[...] = jnp.zeros((G, D_NODE), _f32)

    dn = (((0,), (0,)), ((), ()))
    xsum[...] += lax.dot_general(oh, xn, dn, preferred_element_type=_f32)
    nct[...] += lax.dot_general(oh, jnp.ones((NBLK, D_NODE), _f32), dn,
                                preferred_element_type=_f32)

    @pl.when(i == (N // NBLK) - 1)
    def _fin():
        xmean = xsum[...] / jnp.maximum(nct[...], 1.0)
        uu = u[...]
        gh = jnp.maximum(
            jnp.dot(uu, Wg1[:D_U, :], preferred_element_type=_f32)
            + jnp.dot(xmean, Wg1[D_U:, :], preferred_element_type=_f32)
            + bg1[0],
            0.0)
        un_out[...] = jnp.dot(gh, Wg2[...], preferred_element_type=_f32) + bg2[0]


def _tc3(x, parts, batch3, u, Wn2a, bn2a, Wn2b, bn2b, Wg1, bg1, Wg2, bg2):
    grid = N // NBLK
    return pl.pallas_call(
        _tc3_body,
        grid=(grid,),
        in_specs=[
            pl.BlockSpec((NBLK, D_NODE), lambda i: (i, 0)),
            pl.BlockSpec((NC, NBLK, ACC_W), lambda i: (0, i, 0)),
            pl.BlockSpec((1, 1, NBLK), lambda i: (i, 0, 0)),
            pl.BlockSpec(u.shape, lambda i: (0, 0)),
            pl.BlockSpec(Wn2a.shape, lambda i: (0, 0)),
            pl.BlockSpec(bn2a.shape, lambda i: (0, 0)),
            pl.BlockSpec(Wn2b.shape, lambda i: (0, 0)),
            pl.BlockSpec(bn2b.shape, lambda i: (0, 0)),
            pl.BlockSpec(Wg1.shape, lambda i: (0, 0)),
            pl.BlockSpec(bg1.shape, lambda i: (0, 0)),
            pl.BlockSpec(Wg2.shape, lambda i: (0, 0)),
            pl.BlockSpec(bg2.shape, lambda i: (0, 0)),
        ],
        out_specs=[
            pl.BlockSpec((NBLK, D_NODE), lambda i: (i, 0)),
            pl.BlockSpec((G, D_U), lambda i: (0, 0)),
        ],
        out_shape=[
            jax.ShapeDtypeStruct((N, D_NODE), _f32),
            jax.ShapeDtypeStruct((G, D_U), _f32),
        ],
        scratch_shapes=[
            pltpu.VMEM((G, D_NODE), _f32),
            pltpu.VMEM((G, D_NODE), _f32),
        ],
    )(x, parts, batch3, u, Wn2a, bn2a, Wn2b, bn2b, Wg1, bg1, Wg2, bg2)


def kernel(x, edge_index, edge_attr, u, batch,
           We1, be1, We2, be2,
           Wn1a, bn1a, Wn1b, bn1b,
           Wn2a, bn2a, Wn2b, bn2b,
           Wg1, bg1, Wg2, bg2):
    row = edge_index[0]
    col = edge_index[1]
    pad = E_PAD_H - EH

    def _idx(v, fill):
        return jnp.pad(v, (0, pad), constant_values=fill).reshape(IDX_ROWS_H, 128)

    ridx_a = _idx(row[:EH], 0)
    ridx_b = _idx(row[EH:], 0)
    cg_a = _idx(col[:EH], 0)
    cg_b = _idx(col[EH:], 0)
    cs_a = _idx(col[:EH], N)
    cs_b = _idx(col[EH:], N)
    batch3 = batch.reshape(N // NBLK, 1, NBLK)
    be1_ = be1.reshape(1, -1)
    be2_ = be2.reshape(1, -1)
    bn1a_ = bn1a.reshape(1, -1)
    bn1b_ = bn1b.reshape(1, -1)
    bn2a_ = bn2a.reshape(1, -1)
    bn2b_ = bn2b.reshape(1, -1)
    bg1_ = bg1.reshape(1, -1)
    bg2_ = bg2.reshape(1, -1)

    trow, tcol = _tc1(x, batch3, u, We1, be1_, Wn1a, bn1a_)
    g_a = _sc_gather(trow, tcol, ridx_a, cg_a)
    g_b = _sc_gather(trow, tcol, ridx_b, cg_b)
    en_a, m_a = _tc2(g_a, edge_attr, We1, We2, be2_, Wn1a, Wn1b, bn1b_, 0)
    edge_new, m_b = _tc2(g_b, edge_attr, We1, We2, be2_, Wn1a, Wn1b, bn1b_, 1,
                         en_prev=en_a)
    parts = _sc_scatter(m_a, m_b, cs_a, cs_b)
    x_new, u_new = _tc3(x, parts, batch3, u,
                        Wn2a, bn2a_, Wn2b, bn2b_, Wg1, bg1_, Wg2, bg2_)
    return (x_new, edge_new, u_new)


# speedup vs baseline: 9.0039x; 1.1688x over previous
import functools

import jax
import jax.numpy as jnp
from jax import lax
from jax.experimental import pallas as pl
from jax.experimental.pallas import tpu as pltpu
from jax.experimental.pallas import tpu_sc as plsc

N = 10000
E = 320000
G = 16
D_NODE = 128
D_EDGE = 16
D_U = 16
MSG = 32

NC = 2
NS = 16
NW = NC * NS
EH = E // 2
ROWS_PER_W = 40
IDX_ROWS_H = NW * ROWS_PER_W
E_PAD_H = IDX_ROWS_H * 128
CH = 4
N_SUP = ROWS_PER_W // (2 * CH)
N_ACC = N + 16
ACC_W = 48
ROWS_PER_TILE = N_ACC // NS
NBLK = 1000
EBLK = 3200

_f32 = jnp.float32


def _tc1_body(xb, bb, u, We1, be1, Wn1a, bn1a, trow, tcol):
    xv = xb[...]
    b = bb[0, 0, :]
    oh = (b[:, None] == lax.broadcasted_iota(jnp.int32, (NBLK, G), 1)).astype(_f32)
    w = We1[...]
    ub1 = jnp.dot(u[...], w[2 * D_NODE + D_EDGE:, :], preferred_element_type=_f32)
    rowp = (jnp.dot(xv, w[:D_NODE, :], preferred_element_type=_f32)
            + jnp.dot(oh, ub1, preferred_element_type=_f32) + be1[0])
    xm = jnp.dot(xv, Wn1a[:D_NODE, :], preferred_element_type=_f32) + bn1a[0]
    trow[:, 0:MSG] = rowp
    trow[:, MSG:2 * MSG] = xm
    tcol[...] = jnp.dot(xv, w[D_NODE:2 * D_NODE, :], preferred_element_type=_f32)


def _tc1(x, batch3, u, We1, be1, Wn1a, bn1a):
    grid = N // NBLK
    return pl.pallas_call(
        _tc1_body,
        grid=(grid,),
        in_specs=[
            pl.BlockSpec((NBLK, D_NODE), lambda i: (i, 0)),
            pl.BlockSpec((1, 1, NBLK), lambda i: (i, 0, 0)),
            pl.BlockSpec(u.shape, lambda i: (0, 0)),
            pl.BlockSpec(We1.shape, lambda i: (0, 0)),
            pl.BlockSpec(be1.shape, lambda i: (0, 0)),
            pl.BlockSpec(Wn1a.shape, lambda i: (0, 0)),
            pl.BlockSpec(bn1a.shape, lambda i: (0, 0)),
        ],
        out_specs=[
            pl.BlockSpec((NBLK, 2 * MSG), lambda i: (i, 0)),
            pl.BlockSpec((NBLK, MSG), lambda i: (i, 0)),
        ],
        out_shape=[
            jax.ShapeDtypeStruct((N, 2 * MSG), _f32),
            jax.ShapeDtypeStruct((N, MSG), _f32),
        ],
    )(x, batch3, u, We1, be1, Wn1a, bn1a)


def _sc_gather_body(trow, tcol, ridx, cidx, grow,
                    idxr, idxc, bufr0, bufr1, bufc0, bufc1,
                    semi, semg0, semg1, semw0, semw1):
    c = lax.axis_index("c")
    s = lax.axis_index("s")
    wid = s * NC + c
    rbase = wid * ROWS_PER_W
    ebase = rbase * 128
    cpi1 = pltpu.async_copy(ridx.at[pl.ds(rbase, ROWS_PER_W)], idxr, semi)
    cpi2 = pltpu.async_copy(cidx.at[pl.ds(rbase, ROWS_PER_W)], idxc, semi)
    cpi1.wait()
    cpi2.wait()

    def g_list(k, br, bc, sem):
        r0 = k * CH
        out = []
        for j in range(CH):
            out.append((trow.at[idxr.at[r0 + j]],
                        br.at[pl.ds(j * 128, 128)], sem))
            out.append((tcol.at[idxc.at[r0 + j]],
                        bc.at[pl.ds(j * 128, 128)], sem))
        return out

    def w_list(k, br, bc, sem):
        e0 = ebase + k * CH * 128
        return [(br, grow.at[pl.ds(e0, CH * 128), pl.ds(0, 2 * MSG)], sem),
                (bc, grow.at[pl.ds(e0, CH * 128), pl.ds(2 * MSG, MSG)], sem)]

    def fire(tl):
        for a, b, m in tl:
            pltpu.async_copy(a, b, m)

    def drain(tl):
        for a, b, m in tl:
            pltpu.make_async_copy(a, b, m).wait()

    fire(g_list(0, bufr0, bufc0, semg0))

    def sup(i, carry):
        k0 = 2 * i
        k1 = 2 * i + 1

        @pl.when(i > 0)
        def _():
            drain(w_list(k1, bufr1, bufc1, semw1))

        fire(g_list(k1, bufr1, bufc1, semg1))
        drain(g_list(k0, bufr0, bufc0, semg0))
        fire(w_list(k0, bufr0, bufc0, semw0))
        drain(g_list(k1, bufr1, bufc1, semg1))
        fire(w_list(k1, bufr1, bufc1, semw1))

        @pl.when(i < N_SUP - 1)
        def _():
            drain(w_list(k0, bufr0, bufc0, semw0))
            fire(g_list(k0 + 2, bufr0, bufc0, semg0))

        return carry

    lax.fori_loop(0, N_SUP, sup, 0)
    drain(w_list(0, bufr0, bufc0, semw0))
    drain(w_list(0, bufr1, bufc1, semw1))


def _sc_gather(trow, tcol, ridx, cidx):
    mesh = plsc.VectorSubcoreMesh(core_axis_name="c", subcore_axis_name="s",
                                  num_cores=NC, num_subcores=NS)
    fn = pl.kernel(
        _sc_gather_body,
        out_type=jax.ShapeDtypeStruct((E_PAD_H, 128), _f32),
        mesh=mesh,
        compiler_params=pltpu.CompilerParams(use_tc_tiling_on_sc=False),
        scratch_types=[
            pltpu.VMEM((ROWS_PER_W, 128), jnp.int32),
            pltpu.VMEM((ROWS_PER_W, 128), jnp.int32),
            pltpu.VMEM((CH * 128, 2 * MSG), _f32),
            pltpu.VMEM((CH * 128, 2 * MSG), _f32),
            pltpu.VMEM((CH * 128, MSG), _f32),
            pltpu.VMEM((CH * 128, MSG), _f32),
            pltpu.SemaphoreType.DMA,
            pltpu.SemaphoreType.DMA,
            pltpu.SemaphoreType.DMA,
            pltpu.SemaphoreType.DMA,
            pltpu.SemaphoreType.DMA,
        ],
    )
    return fn(trow, tcol, ridx, cidx)


def _tc2_body(gb, eaTb, We1, We2, be2, Wn1a, Wn1b, bn1b, *rest):
    en_out, m_out = rest[-2], rest[-1]
    g = gb[...]
    dn0 = (((0,), (0,)), ((), ()))
    eap = lax.dot_general(eaTb[...],
                          We1[2 * D_NODE:2 * D_NODE + D_EDGE, :], dn0,
                          preferred_element_type=_f32)
    h1 = jnp.maximum(g[:, 0:MSG] + g[:, 2 * MSG:3 * MSG] + eap, 0.0)
    enT = lax.dot_general(We2[...], h1, (((0,), (1,)), ((), ())),
                          preferred_element_type=_f32) + be2[...]
    en_out[...] = enT
    mh = jnp.maximum(
        g[:, MSG:2 * MSG]
        + lax.dot_general(enT, Wn1a[D_NODE:D_NODE + D_EDGE, :], dn0,
                          preferred_element_type=_f32),
        0.0)
    m = jnp.dot(mh, Wn1b[...], preferred_element_type=_f32) + bn1b[0]
    m_out[:, 0:MSG] = m
    m_out[:, MSG:] = jnp.zeros((EBLK, 128 - MSG), _f32)


def _tc2(g, ea, We1, We2, be2, Wn1a, Wn1b, bn1b, half, en_prev=None):
    grid = EH // EBLK
    off = half * grid
    in_specs = [
        pl.BlockSpec((EBLK, 128), lambda i: (i, 0)),
        pl.BlockSpec((D_EDGE, EBLK), lambda i: (0, i + off)),
        pl.BlockSpec(We1.shape, lambda i: (0, 0)),
        pl.BlockSpec(We2.shape, lambda i: (0, 0)),
        pl.BlockSpec(be2.shape, lambda i: (0, 0)),
        pl.BlockSpec(Wn1a.shape, lambda i: (0, 0)),
        pl.BlockSpec(Wn1b.shape, lambda i: (0, 0)),
        pl.BlockSpec(bn1b.shape, lambda i: (0, 0)),
    ]
    args = [g, ea, We1, We2, be2, Wn1a, Wn1b, bn1b]
    aliases = {}
    if en_prev is not None:
        in_specs.append(pl.BlockSpec(memory_space=pl.ANY))
        args.append(en_prev)
        aliases = {8: 0}
    return pl.pallas_call(
        _tc2_body,
        grid=(grid,),
        in_specs=in_specs,
        out_specs=[
            pl.BlockSpec((D_EDGE, EBLK), lambda i: (0, i + off)),
            pl.BlockSpec((EBLK, 128), lambda i: (i, 0)),
        ],
        out_shape=[
            jax.ShapeDtypeStruct((D_EDGE, E), _f32),
            jax.ShapeDtypeStruct((E_PAD_H, 128), _f32),
        ],
        input_output_aliases=aliases,
    )(*args)


def _sc_scatter_body(m_a, m_b, ca, cb, out_hbm, acc, mbuf0, mbuf1, idxc, zbuf,
                     semz, semi, seml0, seml1, sems0, sems1, semo):
    c = lax.axis_index("c")
    s = lax.axis_index("s")
    wid = s * NC + c
    rbase = wid * ROWS_PER_W
    ebase = rbase * 128

    def zfill(i, carry):
        zbuf[i, pl.ds(0, 16)] = jnp.zeros((16,), _f32)
        zbuf[i, pl.ds(16, 16)] = jnp.zeros((16,), _f32)
        zbuf[i, pl.ds(32, 16)] = jnp.zeros((16,), _f32)
        return carry

    lax.fori_loop(0, ROWS_PER_TILE // 2, zfill, 0)
    row0 = s * ROWS_PER_TILE
    pltpu.async_copy(zbuf, acc.at[pl.ds(row0, ROWS_PER_TILE // 2)], semz).wait()
    pltpu.async_copy(
        zbuf, acc.at[pl.ds(row0 + ROWS_PER_TILE // 2, ROWS_PER_TILE // 2)],
        semz).wait()

    def ofill(i, carry):
        mbuf0[i, pl.ds(MSG, 16)] = jnp.full((16,), 1.0, _f32)
        mbuf1[i, pl.ds(MSG, 16)] = jnp.full((16,), 1.0, _f32)
        return carry

    lax.fori_loop(0, CH * 128, ofill, 0)
    plsc.subcore_barrier()

    def fire(tl, add=False):
        for a, b, m in tl:
            pltpu.async_copy(a, b, m, add=add)

    def drain(tl):
        for a, b, m in tl:
            pltpu.make_async_copy(a, b, m).wait()

    def run_half(m_hbm, cidx):
        pltpu.async_copy(cidx.at[pl.ds(rbase, ROWS_PER_W)], idxc, semi).wait()

        def l_list(k, mb, sem):
            e0 = ebase + k * CH * 128
            return [(m_hbm.at[pl.ds(e0, CH * 128), pl.ds(0, MSG)],
                     mb.at[:, pl.ds(0, MSG)], sem)]

        def s_list(k, mb, sem):
            r0 = k * CH
            return [(mb.at[pl.ds(j * 128, 128)], acc.at[idxc.at[r0 + j]], sem)
                    for j in range(CH)]

        fire(l_list(0, mbuf0, seml0))

        def sup(i, carry):
            k0 = 2 * i
            k1 = 2 * i + 1

            @pl.when(i > 0)
            def _():
                drain(s_list(k1, mbuf1, sems1))

            fire(l_list(k1, mbuf1, seml1))
            drain(l_list(k0, mbuf0, seml0))
            fire(s_list(k0, mbuf0, sems0), add=True)
            drain(l_list(k1, mbuf1, seml1))
            fire(s_list(k1, mbuf1, sems1), add=True)

            @pl.when(i < N_SUP - 1)
            def _():
                drain(s_list(k0, mbuf0, sems0))
                fire(l_list(k0 + 2, mbuf0, seml0))

            return carry

        lax.fori_loop(0, N_SUP, sup, 0)
        drain(s_list(0, mbuf0, sems0))
        drain(s_list(0, mbuf1, sems1))

    run_half(m_a, ca)
    run_half(m_b, cb)
    plsc.subcore_barrier()
    pltpu.async_copy(acc.at[pl.ds(row0, ROWS_PER_TILE)],
                     out_hbm.at[c, pl.ds(row0, ROWS_PER_TILE)], semo).wait()


def _sc_scatter(m_a, m_b, ca, cb):
    mesh = plsc.VectorSubcoreMesh(core_axis_name="c", subcore_axis_name="s",
                                  num_cores=NC, num_subcores=NS)
    fn = pl.kernel(
        _sc_scatter_body,
        out_type=jax.ShapeDtypeStruct((NC, N_ACC, ACC_W), _f32),
        mesh=mesh,
        compiler_params=pltpu.CompilerParams(use_tc_tiling_on_sc=False),
        scratch_types=[
            pltpu.VMEM_SHARED((N_ACC, ACC_W), _f32),
            pltpu.VMEM((CH * 128, ACC_W), _f32),
            pltpu.VMEM((CH * 128, ACC_W), _f32),
            pltpu.VMEM((ROWS_PER_W, 128), jnp.int32),
            pltpu.VMEM((ROWS_PER_TILE // 2, ACC_W), _f32),
            pltpu.SemaphoreType.DMA,
            pltpu.SemaphoreType.DMA,
            pltpu.SemaphoreType.DMA,
            pltpu.SemaphoreType.DMA,
            pltpu.SemaphoreType.DMA,
            pltpu.SemaphoreType.DMA,
            pltpu.SemaphoreType.DMA,
        ],
    )
    return fn(m_a, m_b, ca, cb)


def _tc3_body(xb, pb, bb, u, Wn2a, bn2a, Wn2b, bn2b, Wg1, bg1, Wg2, bg2,
              xn_out, un_out, xsum, nct):
    i = pl.program_id(0)
    p = pb[...]
    ssum = p[0] + p[1]
    agg = ssum[:, 0:MSG] / jnp.maximum(ssum[:, MSG:MSG + 1], 1.0)
    b = bb[0, 0, :]
    oh = (b[:, None] == lax.broadcasted_iota(jnp.int32, (NBLK, G), 1)).astype(_f32)
    wa = Wn2a[...]
    ub2 = jnp.dot(u[...], wa[D_NODE + MSG:, :], preferred_element_type=_f32)
    h = jnp.maximum(
        jnp.dot(xb[...], wa[:D_NODE, :], preferred_element_type=_f32)
        + jnp.dot(agg, wa[D_NODE:D_NODE + MSG, :], preferred_element_type=_f32)
        + jnp.dot(oh, ub2, preferred_element_type=_f32) + bn2a[0],
        0.0)
    xn = jnp.dot(h, Wn2b[...], preferred_element_type=_f32) + bn2b[0]
    xn_out[...] = xn

    @pl.when(i == 0)
    def _init():
        xsum[...] = jnp.zeros((G, D_NODE), _f32)
        nct[...] = jnp.zeros((G, D_NODE), _f32)

    dn = (((0,), (0,)), ((), ()))
    xsum[...] += lax.dot_general(oh, xn, dn, preferred_element_type=_f32)
    nct[...] += lax.dot_general(oh, jnp.ones((NBLK, D_NODE), _f32), dn,
                                preferred_element_type=_f32)

    @pl.when(i == (N // NBLK) - 1)
    def _fin():
        xmean = xsum[...] / jnp.maximum(nct[...], 1.0)
        uu = u[...]
        gh = jnp.maximum(
            jnp.dot(uu, Wg1[:D_U, :], preferred_element_type=_f32)
            + jnp.dot(xmean, Wg1[D_U:, :], preferred_element_type=_f32)
            + bg1[0],
            0.0)
        un_out[...] = jnp.dot(gh, Wg2[...], preferred_element_type=_f32) + bg2[0]


def _tc3(x, parts, batch3, u, Wn2a, bn2a, Wn2b, bn2b, Wg1, bg1, Wg2, bg2):
    grid = N // NBLK
    return pl.pallas_call(
        _tc3_body,
        grid=(grid,),
        in_specs=[
            pl.BlockSpec((NBLK, D_NODE), lambda i: (i, 0)),
            pl.BlockSpec((NC, NBLK, ACC_W), lambda i: (0, i, 0)),
            pl.BlockSpec((1, 1, NBLK), lambda i: (i, 0, 0)),
            pl.BlockSpec(u.shape, lambda i: (0, 0)),
            pl.BlockSpec(Wn2a.shape, lambda i: (0, 0)),
            pl.BlockSpec(bn2a.shape, lambda i: (0, 0)),
            pl.BlockSpec(Wn2b.shape, lambda i: (0, 0)),
            pl.BlockSpec(bn2b.shape, lambda i: (0, 0)),
            pl.BlockSpec(Wg1.shape, lambda i: (0, 0)),
            pl.BlockSpec(bg1.shape, lambda i: (0, 0)),
            pl.BlockSpec(Wg2.shape, lambda i: (0, 0)),
            pl.BlockSpec(bg2.shape, lambda i: (0, 0)),
        ],
        out_specs=[
            pl.BlockSpec((NBLK, D_NODE), lambda i: (i, 0)),
            pl.BlockSpec((G, D_U), lambda i: (0, 0)),
        ],
        out_shape=[
            jax.ShapeDtypeStruct((N, D_NODE), _f32),
            jax.ShapeDtypeStruct((G, D_U), _f32),
        ],
        scratch_shapes=[
            pltpu.VMEM((G, D_NODE), _f32),
            pltpu.VMEM((G, D_NODE), _f32),
        ],
    )(x, parts, batch3, u, Wn2a, bn2a, Wn2b, bn2b, Wg1, bg1, Wg2, bg2)


def kernel(x, edge_index, edge_attr, u, batch,
           We1, be1, We2, be2,
           Wn1a, bn1a, Wn1b, bn1b,
           Wn2a, bn2a, Wn2b, bn2b,
           Wg1, bg1, Wg2, bg2):
    row = edge_index[0]
    col = edge_index[1]
    pad = E_PAD_H - EH

    def _idx(v, fill):
        return jnp.pad(v, (0, pad), constant_values=fill).reshape(IDX_ROWS_H, 128)

    ridx_a = _idx(row[:EH], 0)
    ridx_b = _idx(row[EH:], 0)
    cg_a = _idx(col[:EH], 0)
    cg_b = _idx(col[EH:], 0)
    cs_a = _idx(col[:EH], N)
    cs_b = _idx(col[EH:], N)
    batch3 = batch.reshape(N // NBLK, 1, NBLK)
    be1_ = be1.reshape(1, -1)
    be2_ = be2.reshape(-1, 1)
    bn1a_ = bn1a.reshape(1, -1)
    bn1b_ = bn1b.reshape(1, -1)
    bn2a_ = bn2a.reshape(1, -1)
    bn2b_ = bn2b.reshape(1, -1)
    bg1_ = bg1.reshape(1, -1)
    bg2_ = bg2.reshape(1, -1)

    trow, tcol = _tc1(x, batch3, u, We1, be1_, Wn1a, bn1a_)
    g_a = _sc_gather(trow, tcol, ridx_a, cg_a)
    g_b = _sc_gather(trow, tcol, ridx_b, cg_b)
    eaT = edge_attr.T
    enT_a, m_a = _tc2(g_a, eaT, We1, We2, be2_, Wn1a, Wn1b, bn1b_, 0)
    enT, m_b = _tc2(g_b, eaT, We1, We2, be2_, Wn1a, Wn1b, bn1b_, 1,
                    en_prev=enT_a)
    edge_new = enT.T
    parts = _sc_scatter(m_a, m_b, cs_a, cs_b)
    x_new, u_new = _tc3(x, parts, batch3, u,
                        Wn2a, bn2a_, Wn2b, bn2b_, Wg1, bg1_, Wg2, bg2_)
    return (x_new, edge_new, u_new)
